# Initial kernel scaffold; baseline (speedup 1.0000x reference)
import jax
import jax.numpy as jnp
from jax.experimental import pallas as pl


def kernel(bond_dist, params, atom_types, edge_index):
    raise NotImplementedError("write your pallas kernel here")


# speedup vs baseline: 1.1765x; 1.1765x over previous
import functools

import jax
import jax.numpy as jnp
from jax import lax
from jax.experimental import pallas as pl
from jax.experimental.pallas import tpu as pltpu
from jax.experimental.pallas import tpu_sc as plsc

N = 50000
E = 800000
D = 64
MAX_N = 9
CUTOFF = 5.0
NBLOCKS = 4
NELEM = 89

E_PAD = 819200
ECHUNK = 512
NSUB = ECHUNK // 128
NCORES = 2
NTILES = 16
NWORK = NCORES * NTILES
GCHUNK = 512
FLUSH = 400
NFLUSH = N // FLUSH

_mesh = plsc.VectorSubcoreMesh(core_axis_name="c", subcore_axis_name="s")


@functools.partial(
    pl.kernel,
    out_type=jax.ShapeDtypeStruct((E_PAD, D), jnp.float32),
    mesh=_mesh,
    scratch_types=[
        pltpu.VMEM((GCHUNK // 128, 128), jnp.int32),
        pltpu.VMEM((GCHUNK, D), jnp.float32),
        pltpu.SemaphoreType.DMA,
    ],
    compiler_params=pltpu.CompilerParams(use_tc_tiling_on_sc=False),
)
def _gather_sc(x_hbm, src2d_hbm, g_hbm, idx_v, rows_v, sem):
    cid = lax.axis_index("c")
    sid = lax.axis_index("s")
    wid = sid * NCORES + cid
    chunks = E_PAD // GCHUNK // NWORK
    base_chunk = wid * chunks

    def body(i, _):
        q = base_chunk + i
        e0 = q * GCHUNK
        r0 = q * (GCHUNK // 128)
        pltpu.sync_copy(src2d_hbm.at[pl.ds(r0, GCHUNK // 128)], idx_v)
        descs = []
        for j in range(GCHUNK // 128):
            descs.append(
                pltpu.async_copy(
                    x_hbm.at[idx_v.at[j]],
                    rows_v.at[pl.ds(j * 128, 128)],
                    sem,
                )
            )
        for d in descs:
            d.wait()
        pltpu.sync_copy(rows_v, g_hbm.at[pl.ds(e0, GCHUNK)])
        return 0

    lax.fori_loop(0, chunks, body, 0)


@functools.partial(
    pl.kernel,
    out_type=jax.ShapeDtypeStruct((NCORES, N, D // 2), jnp.float32),
    mesh=_mesh,
    scratch_types=[
        pltpu.VMEM_SHARED((N, D // 2), jnp.float32),
        pltpu.VMEM((NSUB, 128), jnp.int32),
        pltpu.VMEM((ECHUNK, D // 2), jnp.float32),
    ],
    compiler_params=pltpu.CompilerParams(use_tc_tiling_on_sc=False),
)
def _scatter_sc(m2_hbm, dst2d_hbm, zeros_hbm, agg_hbm, acc_sh, idx_v, mrow_v):
    cid = lax.axis_index("c")
    sid = lax.axis_index("s")

    def zero(k, _):
        q = sid + k * NTILES

        @pl.when(q < NFLUSH)
        def _():
            r0 = q * FLUSH
            pltpu.sync_copy(zeros_hbm.at[pl.ds(r0, FLUSH)],
                            acc_sh.at[pl.ds(r0, FLUSH)])
        return 0

    lax.fori_loop(0, (NFLUSH + NTILES - 1) // NTILES, zero, 0)
    plsc.subcore_barrier()

    chunks = E_PAD // ECHUNK // NTILES
    base_chunk = sid * chunks

    def body(i, _):
        q = base_chunk + i
        e0 = q * ECHUNK
        r0 = q * NSUB
        pltpu.sync_copy(dst2d_hbm.at[pl.ds(r0, NSUB)], idx_v)
        pltpu.sync_copy(m2_hbm.at[cid].at[pl.ds(e0, ECHUNK)], mrow_v)
        for j in range(NSUB):
            pltpu.sync_copy(
                mrow_v.at[pl.ds(j * 128, 128)],
                acc_sh.at[idx_v.at[j]],
                add=True,
            )
        return 0

    lax.fori_loop(0, chunks, body, 0)
    plsc.subcore_barrier()

    def flush(k, _):
        q = sid + k * NTILES

        @pl.when(q < NFLUSH)
        def _():
            r0 = q * FLUSH
            pltpu.sync_copy(acc_sh.at[pl.ds(r0, FLUSH)],
                            mrow_v.at[pl.ds(0, FLUSH)])
            pltpu.sync_copy(mrow_v.at[pl.ds(0, FLUSH)],
                            agg_hbm.at[cid].at[pl.ds(r0, FLUSH)])
        return 0

    lax.fori_loop(0, (NFLUSH + NTILES - 1) // NTILES, flush, 0)


_EB = 2048
_NB = 1000


def _msg_body(bd_ref, g_ref, wbw_ref, wbe_ref, bbe_ref, w1_ref, b1_ref,
              m_ref):
    i = pl.program_id(0)
    bd = bd_ref[0, 0, :]
    r = 0.5 + bd * (CUTOFF - 0.5)
    n = 1.0 + jnp.arange(MAX_N, dtype=jnp.int32).astype(jnp.float32)
    sbf = (jnp.sqrt(2.0 / CUTOFF)
           * jnp.sin(r[:, None] * (n[None, :] * (jnp.pi / CUTOFF)))
           / r[:, None])
    env = 0.5 * (jnp.cos(jnp.pi * r / CUTOFF) + 1.0)
    smooth = sbf * env[:, None]
    bw = jnp.dot(smooth, wbw_ref[...], preferred_element_type=jnp.float32)
    bf = jax.nn.silu(
        jnp.dot(smooth, wbe_ref[...], preferred_element_type=jnp.float32)
        + bbe_ref[...])
    h = g_ref[...] * bw + bf
    mm = jnp.dot(h, w1_ref[...], preferred_element_type=jnp.float32) + b1_ref[...]
    m = jax.nn.silu(mm)
    row = i * _EB + lax.broadcasted_iota(jnp.int32, (_EB, 1), 0)
    m = jnp.where(row < E, m, 0.0)
    m_ref[0] = m[:, : D // 2]
    m_ref[1] = m[:, D // 2:]


def _msg_tc(bd3, g, wbw, wbe, bbe2, w1, b12):
    grid = (E_PAD // _EB,)
    return pl.pallas_call(
        _msg_body,
        grid=grid,
        in_specs=[
            pl.BlockSpec((1, 1, _EB), lambda i: (i, 0, 0)),
            pl.BlockSpec((_EB, D), lambda i: (i, 0)),
            pl.BlockSpec((MAX_N, D), lambda i: (0, 0)),
            pl.BlockSpec((MAX_N, D), lambda i: (0, 0)),
            pl.BlockSpec((1, D), lambda i: (0, 0)),
            pl.BlockSpec((D, D), lambda i: (0, 0)),
            pl.BlockSpec((1, D), lambda i: (0, 0)),
        ],
        out_specs=pl.BlockSpec((NCORES, _EB, D // 2), lambda i: (0, i, 0)),
        out_shape=jax.ShapeDtypeStruct((NCORES, E_PAD, D // 2), jnp.float32),
    )(bd3, g, wbw, wbe, bbe2, w1, b12)


def _emb_body(t_ref, emb_ref, x_ref):
    t = t_ref[0, 0, :]
    oh = (t[:, None] == lax.broadcasted_iota(jnp.int32, (_NB, NELEM), 1)
          ).astype(jnp.float32)
    x_ref[...] = jnp.dot(oh, emb_ref[...], preferred_element_type=jnp.float32)


def _emb_tc(t3, emb):
    return pl.pallas_call(
        _emb_body,
        grid=(N // _NB,),
        in_specs=[
            pl.BlockSpec((1, 1, _NB), lambda i: (i, 0, 0)),
            pl.BlockSpec((NELEM, D), lambda i: (0, 0)),
        ],
        out_specs=pl.BlockSpec((_NB, D), lambda i: (i, 0)),
        out_shape=jax.ShapeDtypeStruct((N, D), jnp.float32),
    )(t3, emb)


def _upd_body(x_ref, agg_ref, w2_ref, o_ref):
    a0 = agg_ref[0]
    a1 = agg_ref[1]
    w2 = w2_ref[...]
    y = (jnp.dot(a0, w2[: D // 2, :], preferred_element_type=jnp.float32)
         + jnp.dot(a1, w2[D // 2:, :], preferred_element_type=jnp.float32))
    o_ref[...] = x_ref[...] + y


def _upd_tc(x, agg, w2):
    return pl.pallas_call(
        _upd_body,
        grid=(N // _NB,),
        in_specs=[
            pl.BlockSpec((_NB, D), lambda i: (i, 0)),
            pl.BlockSpec((NCORES, _NB, D // 2), lambda i: (0, i, 0)),
            pl.BlockSpec((D, D), lambda i: (0, 0)),
        ],
        out_specs=pl.BlockSpec((_NB, D), lambda i: (i, 0)),
        out_shape=jax.ShapeDtypeStruct((N, D), jnp.float32),
    )(x, agg, w2)


def _final_body(x_ref, agg_ref, w2_ref, wout_ref, bout_ref, o_ref):
    a0 = agg_ref[0]
    a1 = agg_ref[1]
    w2 = w2_ref[...]
    y = (jnp.dot(a0, w2[: D // 2, :], preferred_element_type=jnp.float32)
         + jnp.dot(a1, w2[D // 2:, :], preferred_element_type=jnp.float32))
    xf = x_ref[...] + y
    e = jnp.sum(xf * wout_ref[...], axis=1, keepdims=True) + bout_ref[0, 0]
    o_ref[...] = e


def _final_tc(x, agg, w2, wout2, bout2):
    return pl.pallas_call(
        _final_body,
        grid=(N // _NB,),
        in_specs=[
            pl.BlockSpec((_NB, D), lambda i: (i, 0)),
            pl.BlockSpec((NCORES, _NB, D // 2), lambda i: (0, i, 0)),
            pl.BlockSpec((D, D), lambda i: (0, 0)),
            pl.BlockSpec((1, D), lambda i: (0, 0)),
            pl.BlockSpec((1, 1), lambda i: (0, 0)),
        ],
        out_specs=pl.BlockSpec((_NB, 1), lambda i: (i, 0)),
        out_shape=jax.ShapeDtypeStruct((N, 1), jnp.float32),
    )(x, agg, w2, wout2, bout2)


def kernel(bond_dist, params, atom_types, edge_index):
    pad = E_PAD - E
    fill = (jnp.arange(pad, dtype=jnp.int32) * 997) % N
    src = jnp.concatenate([edge_index[0].astype(jnp.int32), fill])
    dst = jnp.concatenate([edge_index[1].astype(jnp.int32), fill])
    src2d = src.reshape(E_PAD // 128, 128)
    dst2d = dst.reshape(E_PAD // 128, 128)
    bd3 = jnp.concatenate(
        [bond_dist, jnp.full((pad,), 0.5, jnp.float32)]
    ).reshape(E_PAD // _EB, 1, _EB)
    t3 = atom_types.astype(jnp.int32).reshape(N // _NB, 1, _NB)

    zeros_half = jnp.zeros((N, D // 2), jnp.float32)
    bbe2 = params["b_be"].reshape(1, D)
    wout2 = params["W_out"].reshape(1, D)
    bout2 = params["b_out"].reshape(1, 1).astype(jnp.float32)

    x = _emb_tc(t3, params["atom_emb"])
    for k, blk in enumerate(params["blocks"]):
        g = _gather_sc(x, src2d)
        m2 = _msg_tc(bd3, g, params["W_bw"], params["W_be"], bbe2,
                     blk["W1"], blk["b1"].reshape(1, D))
        agg = _scatter_sc(m2, dst2d, zeros_half)
        if k < NBLOCKS - 1:
            x = _upd_tc(x, agg, blk["W2"])
        else:
            out = _final_tc(x, agg, blk["W2"], wout2, bout2)
    return out.reshape(N)


# speedup vs baseline: 1.8039x; 1.5333x over previous
import functools

import jax
import jax.numpy as jnp
from jax import lax
from jax.experimental import pallas as pl
from jax.experimental.pallas import tpu as pltpu
from jax.experimental.pallas import tpu_sc as plsc

N = 50000
E = 800000
D = 64
MAX_N = 9
CUTOFF = 5.0
NBLOCKS = 4
NELEM = 89

E_PAD = 819200
ECHUNK = 512
NSUB = ECHUNK // 128
NCORES = 2
NTILES = 16
NWORK = NCORES * NTILES
GCHUNK = 512
FLUSH = 400
NFLUSH = N // FLUSH

_mesh = plsc.VectorSubcoreMesh(core_axis_name="c", subcore_axis_name="s")


@functools.partial(
    pl.kernel,
    out_type=jax.ShapeDtypeStruct((E_PAD, D), jnp.float32),
    mesh=_mesh,
    scratch_types=[
        pltpu.VMEM((GCHUNK // 128, 128), jnp.int32),
        pltpu.VMEM((GCHUNK, D), jnp.float32),
        pltpu.SemaphoreType.DMA,
    ],
    compiler_params=pltpu.CompilerParams(use_tc_tiling_on_sc=False),
)
def _gather_sc(x_hbm, src2d_hbm, g_hbm, idx_v, rows_v, sem):
    cid = lax.axis_index("c")
    sid = lax.axis_index("s")
    wid = sid * NCORES + cid
    chunks = E_PAD // GCHUNK // NWORK
    base_chunk = wid * chunks

    def body(i, _):
        q = base_chunk + i
        e0 = q * GCHUNK
        r0 = q * (GCHUNK // 128)
        pltpu.sync_copy(src2d_hbm.at[pl.ds(r0, GCHUNK // 128)], idx_v)
        descs = []
        for j in range(GCHUNK // 128):
            descs.append(
                pltpu.async_copy(
                    x_hbm.at[idx_v.at[j]],
                    rows_v.at[pl.ds(j * 128, 128)],
                    sem,
                )
            )
        for d in descs:
            d.wait()
        pltpu.sync_copy(rows_v, g_hbm.at[pl.ds(e0, GCHUNK)])
        return 0

    lax.fori_loop(0, chunks, body, 0)


@functools.partial(
    pl.kernel,
    out_type=jax.ShapeDtypeStruct((NCORES, N, D // 2), jnp.float32),
    mesh=_mesh,
    scratch_types=[
        pltpu.VMEM_SHARED((N, D // 2), jnp.float32),
        pltpu.VMEM((NSUB, 128), jnp.int32),
        pltpu.VMEM((ECHUNK, D // 2), jnp.float32),
    ],
    compiler_params=pltpu.CompilerParams(use_tc_tiling_on_sc=False),
)
def _scatter_sc(m2_hbm, dst2d_hbm, zeros_hbm, agg_hbm, acc_sh, idx_v, mrow_v):
    cid = lax.axis_index("c")
    sid = lax.axis_index("s")

    def zero(k, _):
        q = sid + k * NTILES

        @pl.when(q < NFLUSH)
        def _():
            r0 = q * FLUSH
            pltpu.sync_copy(zeros_hbm.at[pl.ds(r0, FLUSH)],
                            acc_sh.at[pl.ds(r0, FLUSH)])
        return 0

    lax.fori_loop(0, (NFLUSH + NTILES - 1) // NTILES, zero, 0)
    plsc.subcore_barrier()

    chunks = E_PAD // ECHUNK // NTILES
    base_chunk = sid * chunks

    def body(i, _):
        q = base_chunk + i
        e0 = q * ECHUNK
        r0 = q * NSUB
        pltpu.sync_copy(dst2d_hbm.at[pl.ds(r0, NSUB)], idx_v)
        pltpu.sync_copy(m2_hbm.at[cid].at[pl.ds(e0, ECHUNK)], mrow_v)
        for j in range(NSUB):
            pltpu.sync_copy(
                mrow_v.at[pl.ds(j * 128, 128)],
                acc_sh.at[idx_v.at[j]],
                add=True,
            )
        return 0

    lax.fori_loop(0, chunks, body, 0)
    plsc.subcore_barrier()

    def flush(k, _):
        q = sid + k * NTILES

        @pl.when(q < NFLUSH)
        def _():
            r0 = q * FLUSH
            pltpu.sync_copy(acc_sh.at[pl.ds(r0, FLUSH)],
                            mrow_v.at[pl.ds(0, FLUSH)])
            pltpu.sync_copy(mrow_v.at[pl.ds(0, FLUSH)],
                            agg_hbm.at[cid].at[pl.ds(r0, FLUSH)])
        return 0

    lax.fori_loop(0, (NFLUSH + NTILES - 1) // NTILES, flush, 0)


_EB = 2048
_NB = 1000


def _msg_body(bd_ref, g_ref, wbw_ref, wbe_ref, bbe_ref, w1_ref, b1_ref,
              m_ref):
    i = pl.program_id(0)
    bd = bd_ref[0, 0, :]
    r = 0.5 + bd * (CUTOFF - 0.5)
    theta = r * (jnp.pi / CUTOFF)
    s1 = jnp.sin(theta)
    c1 = jnp.cos(theta)
    env = 0.5 * (c1 + 1.0)
    pref = jnp.sqrt(2.0 / CUTOFF) * env / r
    two_c1 = 2.0 * c1
    rows = [s1 * pref]
    s_prev, s_cur = jnp.zeros_like(s1), s1
    for _ in range(MAX_N - 1):
        s_prev, s_cur = s_cur, two_c1 * s_cur - s_prev
        rows.append(s_cur * pref)
    smooth_t = jnp.concatenate([x[None, :] for x in rows], axis=0)
    cdims = (((0,), (0,)), ((), ()))
    bw = lax.dot_general(smooth_t, wbw_ref[...], cdims,
                         preferred_element_type=jnp.float32)
    bf = jax.nn.silu(
        lax.dot_general(smooth_t, wbe_ref[...], cdims,
                        preferred_element_type=jnp.float32)
        + bbe_ref[...])
    h = g_ref[...] * bw + bf
    mm = jnp.dot(h, w1_ref[...], preferred_element_type=jnp.float32) + b1_ref[...]
    m = jax.nn.silu(mm)
    row = i * _EB + lax.broadcasted_iota(jnp.int32, (_EB, 1), 0)
    m = jnp.where(row < E, m, 0.0)
    m_ref[0] = m[:, : D // 2]
    m_ref[1] = m[:, D // 2:]


def _msg_tc(bd3, g, wbw, wbe, bbe2, w1, b12):
    grid = (E_PAD // _EB,)
    return pl.pallas_call(
        _msg_body,
        grid=grid,
        in_specs=[
            pl.BlockSpec((1, 1, _EB), lambda i: (i, 0, 0)),
            pl.BlockSpec((_EB, D), lambda i: (i, 0)),
            pl.BlockSpec((MAX_N, D), lambda i: (0, 0)),
            pl.BlockSpec((MAX_N, D), lambda i: (0, 0)),
            pl.BlockSpec((1, D), lambda i: (0, 0)),
            pl.BlockSpec((D, D), lambda i: (0, 0)),
            pl.BlockSpec((1, D), lambda i: (0, 0)),
        ],
        out_specs=pl.BlockSpec((NCORES, _EB, D // 2), lambda i: (0, i, 0)),
        out_shape=jax.ShapeDtypeStruct((NCORES, E_PAD, D // 2), jnp.float32),
    )(bd3, g, wbw, wbe, bbe2, w1, b12)


def _emb_body(t_ref, emb_ref, x_ref):
    t = t_ref[0, 0, :]
    oh = (t[:, None] == lax.broadcasted_iota(jnp.int32, (_NB, NELEM), 1)
          ).astype(jnp.float32)
    x_ref[...] = jnp.dot(oh, emb_ref[...], preferred_element_type=jnp.float32)


def _emb_tc(t3, emb):
    return pl.pallas_call(
        _emb_body,
        grid=(N // _NB,),
        in_specs=[
            pl.BlockSpec((1, 1, _NB), lambda i: (i, 0, 0)),
            pl.BlockSpec((NELEM, D), lambda i: (0, 0)),
        ],
        out_specs=pl.BlockSpec((_NB, D), lambda i: (i, 0)),
        out_shape=jax.ShapeDtypeStruct((N, D), jnp.float32),
    )(t3, emb)


def _upd_body(x_ref, agg_ref, w2_ref, o_ref):
    a0 = agg_ref[0]
    a1 = agg_ref[1]
    w2 = w2_ref[...]
    y = (jnp.dot(a0, w2[: D // 2, :], preferred_element_type=jnp.float32)
         + jnp.dot(a1, w2[D // 2:, :], preferred_element_type=jnp.float32))
    o_ref[...] = x_ref[...] + y


def _upd_tc(x, agg, w2):
    return pl.pallas_call(
        _upd_body,
        grid=(N // _NB,),
        in_specs=[
            pl.BlockSpec((_NB, D), lambda i: (i, 0)),
            pl.BlockSpec((NCORES, _NB, D // 2), lambda i: (0, i, 0)),
            pl.BlockSpec((D, D), lambda i: (0, 0)),
        ],
        out_specs=pl.BlockSpec((_NB, D), lambda i: (i, 0)),
        out_shape=jax.ShapeDtypeStruct((N, D), jnp.float32),
    )(x, agg, w2)


def _final_body(x_ref, agg_ref, w2_ref, wout_ref, bout_ref, o_ref):
    a0 = agg_ref[0]
    a1 = agg_ref[1]
    w2 = w2_ref[...]
    y = (jnp.dot(a0, w2[: D // 2, :], preferred_element_type=jnp.float32)
         + jnp.dot(a1, w2[D // 2:, :], preferred_element_type=jnp.float32))
    xf = x_ref[...] + y
    e = jnp.sum(xf * wout_ref[...], axis=1, keepdims=True) + bout_ref[0, 0]
    o_ref[...] = e


def _final_tc(x, agg, w2, wout2, bout2):
    return pl.pallas_call(
        _final_body,
        grid=(N // _NB,),
        in_specs=[
            pl.BlockSpec((_NB, D), lambda i: (i, 0)),
            pl.BlockSpec((NCORES, _NB, D // 2), lambda i: (0, i, 0)),
            pl.BlockSpec((D, D), lambda i: (0, 0)),
            pl.BlockSpec((1, D), lambda i: (0, 0)),
            pl.BlockSpec((1, 1), lambda i: (0, 0)),
        ],
        out_specs=pl.BlockSpec((_NB, 1), lambda i: (i, 0)),
        out_shape=jax.ShapeDtypeStruct((N, 1), jnp.float32),
    )(x, agg, w2, wout2, bout2)


def kernel(bond_dist, params, atom_types, edge_index):
    pad = E_PAD - E
    fill = (jnp.arange(pad, dtype=jnp.int32) * 997) % N
    src = jnp.concatenate([edge_index[0].astype(jnp.int32), fill])
    dst = jnp.concatenate([edge_index[1].astype(jnp.int32), fill])
    src2d = src.reshape(E_PAD // 128, 128)
    dst2d = dst.reshape(E_PAD // 128, 128)
    bd3 = jnp.concatenate(
        [bond_dist, jnp.full((pad,), 0.5, jnp.float32)]
    ).reshape(E_PAD // _EB, 1, _EB)
    t3 = atom_types.astype(jnp.int32).reshape(N // _NB, 1, _NB)

    zeros_half = jnp.zeros((N, D // 2), jnp.float32)
    bbe2 = params["b_be"].reshape(1, D)
    wout2 = params["W_out"].reshape(1, D)
    bout2 = params["b_out"].reshape(1, 1).astype(jnp.float32)

    x = _emb_tc(t3, params["atom_emb"])
    for k, blk in enumerate(params["blocks"]):
        g = _gather_sc(x, src2d)
        m2 = _msg_tc(bd3, g, params["W_bw"], params["W_be"], bbe2,
                     blk["W1"], blk["b1"].reshape(1, D))
        agg = _scatter_sc(m2, dst2d, zeros_half)
        if k < NBLOCKS - 1:
            x = _upd_tc(x, agg, blk["W2"])
        else:
            out = _final_tc(x, agg, blk["W2"], wout2, bout2)
    return out.reshape(N)


# speedup vs baseline: 3.6636x; 2.0309x over previous
import functools

import jax
import jax.numpy as jnp
from jax import lax
from jax.experimental import pallas as pl
from jax.experimental.pallas import tpu as pltpu
from jax.experimental.pallas import tpu_sc as plsc

N = 50000
E = 800000
D = 64
MAX_N = 9
CUTOFF = 5.0
NBLOCKS = 4
NELEM = 89

E_PAD = 819200
E2 = E_PAD // 2
ECHUNK = 512
NSUB = ECHUNK // 128
NCORES = 2
NTILES = 16
NWORK = NCORES * NTILES
GCHUNK = 512
FLUSH = 400
NFLUSH = N // FLUSH

_mesh = plsc.VectorSubcoreMesh(core_axis_name="c", subcore_axis_name="s")


@functools.partial(
    pl.kernel,
    out_type=jax.ShapeDtypeStruct((E_PAD, D), jnp.float32),
    mesh=_mesh,
    scratch_types=[
        pltpu.VMEM((GCHUNK // 128, 128), jnp.int32),
        pltpu.VMEM((GCHUNK, D), jnp.float32),
        pltpu.SemaphoreType.DMA,
    ],
    compiler_params=pltpu.CompilerParams(use_tc_tiling_on_sc=False),
)
def _gather_sc(x_hbm, src2d_hbm, g_hbm, idx_v, rows_v, sem):
    cid = lax.axis_index("c")
    sid = lax.axis_index("s")
    wid = sid * NCORES + cid
    chunks = E_PAD // GCHUNK // NWORK
    base_chunk = wid * chunks

    def body(i, _):
        q = base_chunk + i
        e0 = q * GCHUNK
        r0 = q * (GCHUNK // 128)
        pltpu.sync_copy(src2d_hbm.at[pl.ds(r0, GCHUNK // 128)], idx_v)
        descs = []
        for j in range(GCHUNK // 128):
            descs.append(
                pltpu.async_copy(
                    x_hbm.at[idx_v.at[j]],
                    rows_v.at[pl.ds(j * 128, 128)],
                    sem,
                )
            )
        for d in descs:
            d.wait()
        pltpu.sync_copy(rows_v, g_hbm.at[pl.ds(e0, GCHUNK)])
        return 0

    lax.fori_loop(0, chunks, body, 0)


@functools.partial(
    pl.kernel,
    out_type=jax.ShapeDtypeStruct((N, 128), jnp.float32),
    mesh=_mesh,
    scratch_types=[
        pltpu.VMEM_SHARED((N, D // 2), jnp.float32),
        pltpu.VMEM((NSUB, 128), jnp.int32),
        pltpu.VMEM((ECHUNK, D // 2), jnp.float32),
    ],
    compiler_params=pltpu.CompilerParams(use_tc_tiling_on_sc=False),
)
def _scatter_sc(m_hbm, dst2d_hbm, zeros_hbm, agg_hbm, acc_sh, idx_v, mrow_v):
    cid = lax.axis_index("c")
    sid = lax.axis_index("s")
    c0 = cid * (D // 2)

    def zero(k, _):
        q = sid + k * NTILES

        @pl.when(q < NFLUSH)
        def _():
            r0 = q * FLUSH
            pltpu.sync_copy(zeros_hbm.at[pl.ds(r0, FLUSH)],
                            acc_sh.at[pl.ds(r0, FLUSH)])
        return 0

    lax.fori_loop(0, (NFLUSH + NTILES - 1) // NTILES, zero, 0)
    plsc.subcore_barrier()

    chunks = E_PAD // ECHUNK // NTILES
    base_chunk = sid * chunks

    def body(i, _):
        q = base_chunk + i
        e0 = q * ECHUNK
        r0 = q * NSUB
        pltpu.sync_copy(dst2d_hbm.at[pl.ds(r0, NSUB)], idx_v)
        pltpu.sync_copy(m_hbm.at[pl.ds(e0, ECHUNK), pl.ds(c0, D // 2)], mrow_v)
        for j in range(NSUB):
            pltpu.sync_copy(
                mrow_v.at[pl.ds(j * 128, 128)],
                acc_sh.at[idx_v.at[j]],
                add=True,
            )
        return 0

    lax.fori_loop(0, chunks, body, 0)
    plsc.subcore_barrier()

    def flush(k, _):
        q = sid + k * NTILES

        @pl.when(q < NFLUSH)
        def _():
            r0 = q * FLUSH
            pltpu.sync_copy(acc_sh.at[pl.ds(r0, FLUSH)],
                            agg_hbm.at[pl.ds(r0, FLUSH), pl.ds(c0, D // 2)])
        return 0

    lax.fori_loop(0, (NFLUSH + NTILES - 1) // NTILES, flush, 0)


_EB = 2048
_PB = _EB // 2
_NB = 1000


def _bessel_rows(bd):
    r = 0.5 + bd * (CUTOFF - 0.5)
    theta = r * (jnp.pi / CUTOFF)
    s1 = jnp.sin(theta)
    c1 = jnp.cos(theta)
    env = 0.5 * (c1 + 1.0)
    pref = jnp.sqrt(2.0 / CUTOFF) * env / r
    two_c1 = 2.0 * c1
    rows = [s1 * pref]
    s_prev, s_cur = jnp.zeros_like(s1), s1
    for _ in range(MAX_N - 1):
        s_prev, s_cur = s_cur, two_c1 * s_cur - s_prev
        rows.append(s_cur * pref)
    return rows


def _msg_body(bde_ref, bdo_ref, g_ref, wbw2_ref, wbe2_ref, bbe2_ref,
              w1d_ref, b1d_ref, m_ref):
    i = pl.program_id(0)
    rows = _bessel_rows(bde_ref[0, 0, :]) + _bessel_rows(bdo_ref[0, 0, :])
    smooth_t = jnp.concatenate([x[None, :] for x in rows], axis=0)
    cdims = (((0,), (0,)), ((), ()))
    bw = lax.dot_general(smooth_t, wbw2_ref[...], cdims,
                         preferred_element_type=jnp.float32)
    bf = jax.nn.silu(
        lax.dot_general(smooth_t, wbe2_ref[...], cdims,
                        preferred_element_type=jnp.float32)
        + bbe2_ref[...])
    h = g_ref[...] * bw + bf
    mm = jnp.dot(h, w1d_ref[...], preferred_element_type=jnp.float32) \
        + b1d_ref[...]
    m = jax.nn.silu(mm)
    rowid = i * _PB + lax.broadcasted_iota(jnp.int32, (_PB, 1), 0)
    m = jnp.where(rowid < E // 2, m, 0.0)
    m_ref[...] = m


def _msg_tc(bde3, bdo3, g128, wbw2, wbe2, bbe2, w1d, b1d):
    grid = (E2 // _PB,)
    return pl.pallas_call(
        _msg_body,
        grid=grid,
        in_specs=[
            pl.BlockSpec((1, 1, _PB), lambda i: (i, 0, 0)),
            pl.BlockSpec((1, 1, _PB), lambda i: (i, 0, 0)),
            pl.BlockSpec((_PB, 128), lambda i: (i, 0)),
            pl.BlockSpec((2 * MAX_N, 128), lambda i: (0, 0)),
            pl.BlockSpec((2 * MAX_N, 128), lambda i: (0, 0)),
            pl.BlockSpec((1, 128), lambda i: (0, 0)),
            pl.BlockSpec((128, 128), lambda i: (0, 0)),
            pl.BlockSpec((1, 128), lambda i: (0, 0)),
        ],
        out_specs=pl.BlockSpec((_PB, 128), lambda i: (i, 0)),
        out_shape=jax.ShapeDtypeStruct((E2, 128), jnp.float32),
    )(bde3, bdo3, g128, wbw2, wbe2, bbe2, w1d, b1d)


def _emb_body(t_ref, emb_ref, x_ref):
    t = t_ref[0, 0, :]
    oh = (t[:, None] == lax.broadcasted_iota(jnp.int32, (_NB, NELEM), 1)
          ).astype(jnp.float32)
    x_ref[...] = jnp.dot(oh, emb_ref[...], preferred_element_type=jnp.float32)


def _emb_tc(t3, emb):
    return pl.pallas_call(
        _emb_body,
        grid=(N // _NB,),
        in_specs=[
            pl.BlockSpec((1, 1, _NB), lambda i: (i, 0, 0)),
            pl.BlockSpec((NELEM, D), lambda i: (0, 0)),
        ],
        out_specs=pl.BlockSpec((_NB, D), lambda i: (i, 0)),
        out_shape=jax.ShapeDtypeStruct((N, D), jnp.float32),
    )(t3, emb)


def _upd_body(x_ref, agg_ref, w2_ref, o_ref):
    a = agg_ref[:, :D]
    y = jnp.dot(a, w2_ref[...], preferred_element_type=jnp.float32)
    o_ref[...] = x_ref[...] + y


def _upd_tc(x, agg, w2):
    return pl.pallas_call(
        _upd_body,
        grid=(N // _NB,),
        in_specs=[
            pl.BlockSpec((_NB, D), lambda i: (i, 0)),
            pl.BlockSpec((_NB, 128), lambda i: (i, 0)),
            pl.BlockSpec((D, D), lambda i: (0, 0)),
        ],
        out_specs=pl.BlockSpec((_NB, D), lambda i: (i, 0)),
        out_shape=jax.ShapeDtypeStruct((N, D), jnp.float32),
    )(x, agg, w2)


def _final_body(x_ref, agg_ref, w2_ref, wout_ref, bout_ref, o_ref):
    a = agg_ref[:, :D]
    y = jnp.dot(a, w2_ref[...], preferred_element_type=jnp.float32)
    xf = x_ref[...] + y
    e = jnp.sum(xf * wout_ref[...], axis=1, keepdims=True) + bout_ref[0, 0]
    o_ref[...] = e


def _final_tc(x, agg, w2, wout2, bout2):
    return pl.pallas_call(
        _final_body,
        grid=(N // _NB,),
        in_specs=[
            pl.BlockSpec((_NB, D), lambda i: (i, 0)),
            pl.BlockSpec((_NB, 128), lambda i: (i, 0)),
            pl.BlockSpec((D, D), lambda i: (0, 0)),
            pl.BlockSpec((1, D), lambda i: (0, 0)),
            pl.BlockSpec((1, 1), lambda i: (0, 0)),
        ],
        out_specs=pl.BlockSpec((_NB, 1), lambda i: (i, 0)),
        out_shape=jax.ShapeDtypeStruct((N, 1), jnp.float32),
    )(x, agg, w2, wout2, bout2)


def _block_diag2(w):
    z = jnp.zeros((w.shape[0] * 2, w.shape[1] * 2), jnp.float32)
    z = z.at[: w.shape[0], : w.shape[1]].set(w)
    return z.at[w.shape[0]:, w.shape[1]:].set(w)


def kernel(bond_dist, params, atom_types, edge_index):
    pad = E_PAD - E
    fill = (jnp.arange(pad, dtype=jnp.int32) * 997) % N
    src = jnp.concatenate([edge_index[0].astype(jnp.int32), fill])
    dst = jnp.concatenate([edge_index[1].astype(jnp.int32), fill])
    src2d = src.reshape(E_PAD // 128, 128)
    dst2d = dst.reshape(E_PAD // 128, 128)
    bdp = jnp.concatenate(
        [bond_dist, jnp.full((pad,), 0.5, jnp.float32)]
    ).reshape(E2, 2)
    bde3 = bdp[:, 0].reshape(E2 // _PB, 1, _PB)
    bdo3 = bdp[:, 1].reshape(E2 // _PB, 1, _PB)
    t3 = atom_types.astype(jnp.int32).reshape(N // _NB, 1, _NB)

    zeros_half = jnp.zeros((N, D // 2), jnp.float32)
    wbw2 = _block_diag2(params["W_bw"])
    wbe2 = _block_diag2(params["W_be"])
    bbe2 = jnp.concatenate([params["b_be"], params["b_be"]]).reshape(1, 128)
    wout2 = params["W_out"].reshape(1, D)
    bout2 = params["b_out"].reshape(1, 1).astype(jnp.float32)

    x = _emb_tc(t3, params["atom_emb"])
    for k, blk in enumerate(params["blocks"]):
        g = _gather_sc(x, src2d)
        g128 = g.reshape(E2, 128)
        w1d = _block_diag2(blk["W1"])
        b1d = jnp.concatenate([blk["b1"], blk["b1"]]).reshape(1, 128)
        m_pair = _msg_tc(bde3, bdo3, g128, wbw2, wbe2, bbe2, w1d, b1d)
        m64 = m_pair.reshape(E_PAD, D)
        agg = _scatter_sc(m64, dst2d, zeros_half)
        if k < NBLOCKS - 1:
            x = _upd_tc(x, agg, blk["W2"])
        else:
            out = _final_tc(x, agg, blk["W2"], wout2, bout2)
    return out.reshape(N)


# speedup vs baseline: 4.1673x; 1.1375x over previous
import functools

import jax
import jax.numpy as jnp
from jax import lax
from jax.experimental import pallas as pl
from jax.experimental.pallas import tpu as pltpu
from jax.experimental.pallas import tpu_sc as plsc

N = 50000
E = 800000
D = 64
MAX_N = 9
CUTOFF = 5.0
NBLOCKS = 4
NELEM = 89

E_PAD = 819200
E2 = E_PAD // 2
ECHUNK = 256
NSUB = ECHUNK // 128
NCORES = 2
NTILES = 16
NWORK = NCORES * NTILES
GCHUNK = 512
FLUSH = 400
NFLUSH = N // FLUSH

_mesh = plsc.VectorSubcoreMesh(core_axis_name="c", subcore_axis_name="s")


@functools.partial(
    pl.kernel,
    out_type=jax.ShapeDtypeStruct((E_PAD, D), jnp.float32),
    mesh=_mesh,
    scratch_types=[
        pltpu.VMEM((GCHUNK // 128, 128), jnp.int32),
        pltpu.VMEM((GCHUNK, D), jnp.float32),
        pltpu.SemaphoreType.DMA,
    ],
    compiler_params=pltpu.CompilerParams(use_tc_tiling_on_sc=False),
)
def _gather_sc(x_hbm, src2d_hbm, g_hbm, idx_v, rows_v, sem):
    cid = lax.axis_index("c")
    sid = lax.axis_index("s")
    wid = sid * NCORES + cid
    chunks = E_PAD // GCHUNK // NWORK
    base_chunk = wid * chunks

    def body(i, _):
        q = base_chunk + i
        e0 = q * GCHUNK
        r0 = q * (GCHUNK // 128)
        pltpu.sync_copy(src2d_hbm.at[pl.ds(r0, GCHUNK // 128)], idx_v)
        descs = []
        for j in range(GCHUNK // 128):
            descs.append(
                pltpu.async_copy(
                    x_hbm.at[idx_v.at[j]],
                    rows_v.at[pl.ds(j * 128, 128)],
                    sem,
                )
            )
        for d in descs:
            d.wait()
        pltpu.sync_copy(rows_v, g_hbm.at[pl.ds(e0, GCHUNK)])
        return 0

    lax.fori_loop(0, chunks, body, 0)


@functools.partial(
    pl.kernel,
    out_type=jax.ShapeDtypeStruct((N, 128), jnp.float32),
    mesh=_mesh,
    scratch_types=[
        pltpu.VMEM_SHARED((N, D // 2), jnp.float32),
        pltpu.VMEM((NSUB, 128), jnp.int32),
        pltpu.VMEM((NSUB, 128), jnp.int32),
        pltpu.VMEM((ECHUNK, D // 2), jnp.float32),
        pltpu.VMEM((ECHUNK, D // 2), jnp.float32),
        pltpu.SemaphoreType.DMA,
        pltpu.SemaphoreType.DMA,
        pltpu.SemaphoreType.DMA,
        pltpu.SemaphoreType.DMA,
        pltpu.SemaphoreType.DMA,
    ],
    compiler_params=pltpu.CompilerParams(use_tc_tiling_on_sc=False),
)
def _scatter_sc(m_hbm, dst2d_hbm, zeros_hbm, agg_hbm, acc_sh,
                idx_v0, idx_v1, mrow_v0, mrow_v1,
                sem_i0, sem_i1, sem_m0, sem_m1, sem_s):
    cid = lax.axis_index("c")
    sid = lax.axis_index("s")
    c0 = cid * (D // 2)
    idx_v = (idx_v0, idx_v1)
    mrow_v = (mrow_v0, mrow_v1)
    sem_i = (sem_i0, sem_i1)
    sem_m = (sem_m0, sem_m1)

    def zero(k, _):
        q = sid + k * NTILES

        @pl.when(q < NFLUSH)
        def _():
            r0 = q * FLUSH
            pltpu.sync_copy(zeros_hbm.at[pl.ds(r0, FLUSH)],
                            acc_sh.at[pl.ds(r0, FLUSH)])
        return 0

    lax.fori_loop(0, (NFLUSH + NTILES - 1) // NTILES, zero, 0)
    plsc.subcore_barrier()

    chunks = E_PAD // ECHUNK // NTILES
    base_chunk = sid * chunks

    def _fetch(q, b):
        e0 = q * ECHUNK
        r0 = q * NSUB
        pltpu.async_copy(dst2d_hbm.at[pl.ds(r0, NSUB)], idx_v[b], sem_i[b])
        pltpu.async_copy(m_hbm.at[pl.ds(e0, ECHUNK), pl.ds(c0, D // 2)],
                         mrow_v[b], sem_m[b])

    for b in range(2):
        _fetch(base_chunk + b, b)

    def body(i2, _):
        for b in range(2):
            i = 2 * i2 + b
            q = base_chunk + i
            pltpu.make_async_copy(
                dst2d_hbm.at[pl.ds(q * NSUB, NSUB)], idx_v[b], sem_i[b]
            ).wait()
            pltpu.make_async_copy(
                m_hbm.at[pl.ds(q * ECHUNK, ECHUNK), pl.ds(c0, D // 2)],
                mrow_v[b], sem_m[b],
            ).wait()
            descs = []
            for j in range(NSUB):
                descs.append(pltpu.async_copy(
                    mrow_v[b].at[pl.ds(j * 128, 128)],
                    acc_sh.at[idx_v[b].at[j]],
                    sem_s, add=True,
                ))
            for d in descs:
                d.wait()

            @pl.when(i + 2 < chunks)
            def _():
                _fetch(q + 2, b)
        return 0

    lax.fori_loop(0, chunks // 2, body, 0)
    plsc.subcore_barrier()

    def flush(k, _):
        q = sid + k * NTILES

        @pl.when(q < NFLUSH)
        def _():
            r0 = q * FLUSH
            pltpu.sync_copy(acc_sh.at[pl.ds(r0, FLUSH)],
                            agg_hbm.at[pl.ds(r0, FLUSH), pl.ds(c0, D // 2)])
        return 0

    lax.fori_loop(0, (NFLUSH + NTILES - 1) // NTILES, flush, 0)


_EB = 2048
_PB = _EB // 2
_NB = 1000


def _bessel_rows(bd):
    r = 0.5 + bd * (CUTOFF - 0.5)
    theta = r * (jnp.pi / CUTOFF)
    s1 = jnp.sin(theta)
    c1 = jnp.cos(theta)
    env = 0.5 * (c1 + 1.0)
    pref = jnp.sqrt(2.0 / CUTOFF) * env / r
    two_c1 = 2.0 * c1
    rows = [s1 * pref]
    s_prev, s_cur = jnp.zeros_like(s1), s1
    for _ in range(MAX_N - 1):
        s_prev, s_cur = s_cur, two_c1 * s_cur - s_prev
        rows.append(s_cur * pref)
    return rows


def _msg_body(bde_ref, bdo_ref, g_ref, wbw2_ref, wbe2_ref, bbe2_ref,
              w1d_ref, b1d_ref, m_ref):
    i = pl.program_id(0)
    rows = _bessel_rows(bde_ref[0, 0, :]) + _bessel_rows(bdo_ref[0, 0, :])
    smooth_t = jnp.concatenate([x[None, :] for x in rows], axis=0)
    cdims = (((0,), (0,)), ((), ()))
    bw = lax.dot_general(smooth_t, wbw2_ref[...], cdims,
                         preferred_element_type=jnp.float32)
    bf = jax.nn.silu(
        lax.dot_general(smooth_t, wbe2_ref[...], cdims,
                        preferred_element_type=jnp.float32)
        + bbe2_ref[...])
    h = g_ref[...] * bw + bf
    mm = jnp.dot(h, w1d_ref[...], preferred_element_type=jnp.float32) \
        + b1d_ref[...]
    m = jax.nn.silu(mm)
    rowid = i * _PB + lax.broadcasted_iota(jnp.int32, (_PB, 1), 0)
    m = jnp.where(rowid < E // 2, m, 0.0)
    m_ref[...] = m


def _msg_tc(bde3, bdo3, g128, wbw2, wbe2, bbe2, w1d, b1d):
    grid = (E2 // _PB,)
    return pl.pallas_call(
        _msg_body,
        grid=grid,
        in_specs=[
            pl.BlockSpec((1, 1, _PB), lambda i: (i, 0, 0)),
            pl.BlockSpec((1, 1, _PB), lambda i: (i, 0, 0)),
            pl.BlockSpec((_PB, 128), lambda i: (i, 0)),
            pl.BlockSpec((2 * MAX_N, 128), lambda i: (0, 0)),
            pl.BlockSpec((2 * MAX_N, 128), lambda i: (0, 0)),
            pl.BlockSpec((1, 128), lambda i: (0, 0)),
            pl.BlockSpec((128, 128), lambda i: (0, 0)),
            pl.BlockSpec((1, 128), lambda i: (0, 0)),
        ],
        out_specs=pl.BlockSpec((_PB, 128), lambda i: (i, 0)),
        out_shape=jax.ShapeDtypeStruct((E2, 128), jnp.float32),
    )(bde3, bdo3, g128, wbw2, wbe2, bbe2, w1d, b1d)


def _emb_body(t_ref, emb_ref, x_ref):
    t = t_ref[0, 0, :]
    oh = (t[:, None] == lax.broadcasted_iota(jnp.int32, (_NB, NELEM), 1)
          ).astype(jnp.float32)
    x_ref[...] = jnp.dot(oh, emb_ref[...], preferred_element_type=jnp.float32)


def _emb_tc(t3, emb):
    return pl.pallas_call(
        _emb_body,
        grid=(N // _NB,),
        in_specs=[
            pl.BlockSpec((1, 1, _NB), lambda i: (i, 0, 0)),
            pl.BlockSpec((NELEM, D), lambda i: (0, 0)),
        ],
        out_specs=pl.BlockSpec((_NB, D), lambda i: (i, 0)),
        out_shape=jax.ShapeDtypeStruct((N, D), jnp.float32),
    )(t3, emb)


def _upd_body(x_ref, agg_ref, w2_ref, o_ref):
    a = agg_ref[:, :D]
    y = jnp.dot(a, w2_ref[...], preferred_element_type=jnp.float32)
    o_ref[...] = x_ref[...] + y


def _upd_tc(x, agg, w2):
    return pl.pallas_call(
        _upd_body,
        grid=(N // _NB,),
        in_specs=[
            pl.BlockSpec((_NB, D), lambda i: (i, 0)),
            pl.BlockSpec((_NB, 128), lambda i: (i, 0)),
            pl.BlockSpec((D, D), lambda i: (0, 0)),
        ],
        out_specs=pl.BlockSpec((_NB, D), lambda i: (i, 0)),
        out_shape=jax.ShapeDtypeStruct((N, D), jnp.float32),
    )(x, agg, w2)


def _final_body(x_ref, agg_ref, w2_ref, wout_ref, bout_ref, o_ref):
    a = agg_ref[:, :D]
    y = jnp.dot(a, w2_ref[...], preferred_element_type=jnp.float32)
    xf = x_ref[...] + y
    e = jnp.sum(xf * wout_ref[...], axis=1, keepdims=True) + bout_ref[0, 0]
    o_ref[...] = e


def _final_tc(x, agg, w2, wout2, bout2):
    return pl.pallas_call(
        _final_body,
        grid=(N // _NB,),
        in_specs=[
            pl.BlockSpec((_NB, D), lambda i: (i, 0)),
            pl.BlockSpec((_NB, 128), lambda i: (i, 0)),
            pl.BlockSpec((D, D), lambda i: (0, 0)),
            pl.BlockSpec((1, D), lambda i: (0, 0)),
            pl.BlockSpec((1, 1), lambda i: (0, 0)),
        ],
        out_specs=pl.BlockSpec((_NB, 1), lambda i: (i, 0)),
        out_shape=jax.ShapeDtypeStruct((N, 1), jnp.float32),
    )(x, agg, w2, wout2, bout2)


def _block_diag2(w):
    z = jnp.zeros((w.shape[0] * 2, w.shape[1] * 2), jnp.float32)
    z = z.at[: w.shape[0], : w.shape[1]].set(w)
    return z.at[w.shape[0]:, w.shape[1]:].set(w)


def kernel(bond_dist, params, atom_types, edge_index):
    pad = E_PAD - E
    fill = (jnp.arange(pad, dtype=jnp.int32) * 997) % N
    src = jnp.concatenate([edge_index[0].astype(jnp.int32), fill])
    dst = jnp.concatenate([edge_index[1].astype(jnp.int32), fill])
    src2d = src.reshape(E_PAD // 128, 128)
    dst2d = dst.reshape(E_PAD // 128, 128)
    bdp = jnp.concatenate(
        [bond_dist, jnp.full((pad,), 0.5, jnp.float32)]
    ).reshape(E2, 2)
    bde3 = bdp[:, 0].reshape(E2 // _PB, 1, _PB)
    bdo3 = bdp[:, 1].reshape(E2 // _PB, 1, _PB)
    t3 = atom_types.astype(jnp.int32).reshape(N // _NB, 1, _NB)

    zeros_half = jnp.zeros((N, D // 2), jnp.float32)
    wbw2 = _block_diag2(params["W_bw"])
    wbe2 = _block_diag2(params["W_be"])
    bbe2 = jnp.concatenate([params["b_be"], params["b_be"]]).reshape(1, 128)
    wout2 = params["W_out"].reshape(1, D)
    bout2 = params["b_out"].reshape(1, 1).astype(jnp.float32)

    x = _emb_tc(t3, params["atom_emb"])
    for k, blk in enumerate(params["blocks"]):
        g = _gather_sc(x, src2d)
        g128 = g.reshape(E2, 128)
        w1d = _block_diag2(blk["W1"])
        b1d = jnp.concatenate([blk["b1"], blk["b1"]]).reshape(1, 128)
        m_pair = _msg_tc(bde3, bdo3, g128, wbw2, wbe2, bbe2, w1d, b1d)
        m64 = m_pair.reshape(E_PAD, D)
        agg = _scatter_sc(m64, dst2d, zeros_half)
        if k < NBLOCKS - 1:
            x = _upd_tc(x, agg, blk["W2"])
        else:
            out = _final_tc(x, agg, blk["W2"], wout2, bout2)
    return out.reshape(N)


# speedup vs baseline: 4.3150x; 1.0354x over previous
import functools

import jax
import jax.numpy as jnp
from jax import lax
from jax.experimental import pallas as pl
from jax.experimental.pallas import tpu as pltpu
from jax.experimental.pallas import tpu_sc as plsc

N = 50000
E = 800000
D = 64
MAX_N = 9
CUTOFF = 5.0
NBLOCKS = 4
NELEM = 89

E_PAD = 819200
E2 = E_PAD // 2
ECHUNK = 256
NSUB = ECHUNK // 128
NCORES = 2
NTILES = 16
NWORK = NCORES * NTILES
GCHUNK = 512
FLUSH = 400
NFLUSH = N // FLUSH

_mesh = plsc.VectorSubcoreMesh(core_axis_name="c", subcore_axis_name="s")


@functools.partial(
    pl.kernel,
    out_type=jax.ShapeDtypeStruct((E_PAD, D), jnp.float32),
    mesh=_mesh,
    scratch_types=[
        pltpu.VMEM((GCHUNK // 128, 128), jnp.int32),
        pltpu.VMEM((GCHUNK // 128, 128), jnp.int32),
        pltpu.VMEM((GCHUNK, D), jnp.float32),
        pltpu.VMEM((GCHUNK, D), jnp.float32),
        pltpu.SemaphoreType.DMA,
        pltpu.SemaphoreType.DMA,
        pltpu.SemaphoreType.DMA,
        pltpu.SemaphoreType.DMA,
        pltpu.SemaphoreType.DMA,
    ],
    compiler_params=pltpu.CompilerParams(use_tc_tiling_on_sc=False),
)
def _gather_sc(x_hbm, src2d_hbm, g_hbm, idx_v0, idx_v1, rows_v0, rows_v1,
               sem_i0, sem_i1, sem_w0, sem_w1, sem_g):
    cid = lax.axis_index("c")
    sid = lax.axis_index("s")
    wid = sid * NCORES + cid
    chunks = E_PAD // GCHUNK // NWORK
    base_chunk = wid * chunks
    idx_v = (idx_v0, idx_v1)
    rows_v = (rows_v0, rows_v1)
    sem_i = (sem_i0, sem_i1)
    sem_w = (sem_w0, sem_w1)
    nsub = GCHUNK // 128

    def _fetch_idx(q, b):
        pltpu.async_copy(src2d_hbm.at[pl.ds(q * nsub, nsub)], idx_v[b],
                         sem_i[b])

    def _process(q, b, first):
        pltpu.make_async_copy(
            src2d_hbm.at[pl.ds(q * nsub, nsub)], idx_v[b], sem_i[b]
        ).wait()
        if not first:
            pltpu.make_async_copy(
                rows_v[b], g_hbm.at[pl.ds(q * GCHUNK, GCHUNK)], sem_w[b]
            ).wait()
        descs = []
        for j in range(nsub):
            descs.append(pltpu.async_copy(
                x_hbm.at[idx_v[b].at[j]],
                rows_v[b].at[pl.ds(j * 128, 128)],
                sem_g,
            ))
        for d in descs:
            d.wait()
        pltpu.async_copy(rows_v[b], g_hbm.at[pl.ds(q * GCHUNK, GCHUNK)],
                         sem_w[b])

        @pl.when(q + 2 < base_chunk + chunks)
        def _():
            _fetch_idx(q + 2, b)

    for b in range(2):
        _fetch_idx(base_chunk + b, b)
    for b in range(2):
        _process(base_chunk + b, b, True)

    def body(i2, _):
        for b in range(2):
            _process(base_chunk + 2 + 2 * i2 + b, b, False)
        return 0

    lax.fori_loop(0, (chunks - 2) // 2, body, 0)
    for b in range(2):
        pltpu.make_async_copy(
            rows_v[b], g_hbm.at[pl.ds(base_chunk * GCHUNK, GCHUNK)], sem_w[b]
        ).wait()


@functools.partial(
    pl.kernel,
    out_type=jax.ShapeDtypeStruct((N, 128), jnp.float32),
    mesh=_mesh,
    scratch_types=[
        pltpu.VMEM_SHARED((N, D // 2), jnp.float32),
        pltpu.VMEM((NSUB, 128), jnp.int32),
        pltpu.VMEM((NSUB, 128), jnp.int32),
        pltpu.VMEM((ECHUNK, D // 2), jnp.float32),
        pltpu.VMEM((ECHUNK, D // 2), jnp.float32),
        pltpu.SemaphoreType.DMA,
        pltpu.SemaphoreType.DMA,
        pltpu.SemaphoreType.DMA,
        pltpu.SemaphoreType.DMA,
        pltpu.SemaphoreType.DMA,
    ],
    compiler_params=pltpu.CompilerParams(use_tc_tiling_on_sc=False),
)
def _scatter_sc(m_hbm, dst2d_hbm, zeros_hbm, agg_hbm, acc_sh,
                idx_v0, idx_v1, mrow_v0, mrow_v1,
                sem_i0, sem_i1, sem_m0, sem_m1, sem_s):
    cid = lax.axis_index("c")
    sid = lax.axis_index("s")
    c0 = cid * (D // 2)
    idx_v = (idx_v0, idx_v1)
    mrow_v = (mrow_v0, mrow_v1)
    sem_i = (sem_i0, sem_i1)
    sem_m = (sem_m0, sem_m1)

    def zero(k, _):
        q = sid + k * NTILES

        @pl.when(q < NFLUSH)
        def _():
            r0 = q * FLUSH
            pltpu.sync_copy(zeros_hbm.at[pl.ds(r0, FLUSH)],
                            acc_sh.at[pl.ds(r0, FLUSH)])
        return 0

    lax.fori_loop(0, (NFLUSH + NTILES - 1) // NTILES, zero, 0)
    plsc.subcore_barrier()

    chunks = E_PAD // ECHUNK // NTILES
    base_chunk = sid * chunks

    def _fetch(q, b):
        e0 = q * ECHUNK
        r0 = q * NSUB
        pltpu.async_copy(dst2d_hbm.at[pl.ds(r0, NSUB)], idx_v[b], sem_i[b])
        pltpu.async_copy(m_hbm.at[pl.ds(e0, ECHUNK), pl.ds(c0, D // 2)],
                         mrow_v[b], sem_m[b])

    for b in range(2):
        _fetch(base_chunk + b, b)

    def body(i2, _):
        for b in range(2):
            i = 2 * i2 + b
            q = base_chunk + i
            pltpu.make_async_copy(
                dst2d_hbm.at[pl.ds(q * NSUB, NSUB)], idx_v[b], sem_i[b]
            ).wait()
            pltpu.make_async_copy(
                m_hbm.at[pl.ds(q * ECHUNK, ECHUNK), pl.ds(c0, D // 2)],
                mrow_v[b], sem_m[b],
            ).wait()
            descs = []
            for j in range(NSUB):
                descs.append(pltpu.async_copy(
                    mrow_v[b].at[pl.ds(j * 128, 128)],
                    acc_sh.at[idx_v[b].at[j]],
                    sem_s, add=True,
                ))
            for d in descs:
                d.wait()

            @pl.when(i + 2 < chunks)
            def _():
                _fetch(q + 2, b)
        return 0

    lax.fori_loop(0, chunks // 2, body, 0)
    plsc.subcore_barrier()

    def flush(k, _):
        q = sid + k * NTILES

        @pl.when(q < NFLUSH)
        def _():
            r0 = q * FLUSH
            pltpu.sync_copy(acc_sh.at[pl.ds(r0, FLUSH)],
                            agg_hbm.at[pl.ds(r0, FLUSH), pl.ds(c0, D // 2)])
        return 0

    lax.fori_loop(0, (NFLUSH + NTILES - 1) // NTILES, flush, 0)


_EB = 2048
_PB = _EB // 2
_NB = 1000


def _bessel_rows(bd):
    r = 0.5 + bd * (CUTOFF - 0.5)
    theta = r * (jnp.pi / CUTOFF)
    s1 = jnp.sin(theta)
    c1 = jnp.cos(theta)
    env = 0.5 * (c1 + 1.0)
    pref = jnp.sqrt(2.0 / CUTOFF) * env / r
    two_c1 = 2.0 * c1
    rows = [s1 * pref]
    s_prev, s_cur = jnp.zeros_like(s1), s1
    for _ in range(MAX_N - 1):
        s_prev, s_cur = s_cur, two_c1 * s_cur - s_prev
        rows.append(s_cur * pref)
    return rows


def _msg_body(bde_ref, bdo_ref, g_ref, wbw2_ref, wbe2_ref, bbe2_ref,
              w1d_ref, b1d_ref, m_ref):
    i = pl.program_id(0)
    rows = _bessel_rows(bde_ref[0, 0, :]) + _bessel_rows(bdo_ref[0, 0, :])
    smooth_t = jnp.concatenate([x[None, :] for x in rows], axis=0)
    cdims = (((0,), (0,)), ((), ()))
    bw = lax.dot_general(smooth_t, wbw2_ref[...], cdims,
                         preferred_element_type=jnp.float32)
    bf = jax.nn.silu(
        lax.dot_general(smooth_t, wbe2_ref[...], cdims,
                        preferred_element_type=jnp.float32)
        + bbe2_ref[...])
    h = g_ref[...] * bw + bf
    mm = jnp.dot(h, w1d_ref[...], preferred_element_type=jnp.float32) \
        + b1d_ref[...]
    m = jax.nn.silu(mm)
    rowid = i * _PB + lax.broadcasted_iota(jnp.int32, (_PB, 1), 0)
    m = jnp.where(rowid < E // 2, m, 0.0)
    m_ref[...] = m


def _msg_tc(bde3, bdo3, g128, wbw2, wbe2, bbe2, w1d, b1d):
    grid = (E2 // _PB,)
    return pl.pallas_call(
        _msg_body,
        grid=grid,
        in_specs=[
            pl.BlockSpec((1, 1, _PB), lambda i: (i, 0, 0)),
            pl.BlockSpec((1, 1, _PB), lambda i: (i, 0, 0)),
            pl.BlockSpec((_PB, 128), lambda i: (i, 0)),
            pl.BlockSpec((2 * MAX_N, 128), lambda i: (0, 0)),
            pl.BlockSpec((2 * MAX_N, 128), lambda i: (0, 0)),
            pl.BlockSpec((1, 128), lambda i: (0, 0)),
            pl.BlockSpec((128, 128), lambda i: (0, 0)),
            pl.BlockSpec((1, 128), lambda i: (0, 0)),
        ],
        out_specs=pl.BlockSpec((_PB, 128), lambda i: (i, 0)),
        out_shape=jax.ShapeDtypeStruct((E2, 128), jnp.float32),
    )(bde3, bdo3, g128, wbw2, wbe2, bbe2, w1d, b1d)


def _emb_body(t_ref, emb_ref, x_ref):
    t = t_ref[0, 0, :]
    oh = (t[:, None] == lax.broadcasted_iota(jnp.int32, (_NB, NELEM), 1)
          ).astype(jnp.float32)
    x_ref[...] = jnp.dot(oh, emb_ref[...], preferred_element_type=jnp.float32)


def _emb_tc(t3, emb):
    return pl.pallas_call(
        _emb_body,
        grid=(N // _NB,),
        in_specs=[
            pl.BlockSpec((1, 1, _NB), lambda i: (i, 0, 0)),
            pl.BlockSpec((NELEM, D), lambda i: (0, 0)),
        ],
        out_specs=pl.BlockSpec((_NB, D), lambda i: (i, 0)),
        out_shape=jax.ShapeDtypeStruct((N, D), jnp.float32),
    )(t3, emb)


def _upd_body(x_ref, agg_ref, w2_ref, o_ref):
    a = agg_ref[:, :D]
    y = jnp.dot(a, w2_ref[...], preferred_element_type=jnp.float32)
    o_ref[...] = x_ref[...] + y


def _upd_tc(x, agg, w2):
    return pl.pallas_call(
        _upd_body,
        grid=(N // _NB,),
        in_specs=[
            pl.BlockSpec((_NB, D), lambda i: (i, 0)),
            pl.BlockSpec((_NB, 128), lambda i: (i, 0)),
            pl.BlockSpec((D, D), lambda i: (0, 0)),
        ],
        out_specs=pl.BlockSpec((_NB, D), lambda i: (i, 0)),
        out_shape=jax.ShapeDtypeStruct((N, D), jnp.float32),
    )(x, agg, w2)


def _final_body(x_ref, agg_ref, w2_ref, wout_ref, bout_ref, o_ref):
    a = agg_ref[:, :D]
    y = jnp.dot(a, w2_ref[...], preferred_element_type=jnp.float32)
    xf = x_ref[...] + y
    e = jnp.sum(xf * wout_ref[...], axis=1, keepdims=True) + bout_ref[0, 0]
    o_ref[...] = e


def _final_tc(x, agg, w2, wout2, bout2):
    return pl.pallas_call(
        _final_body,
        grid=(N // _NB,),
        in_specs=[
            pl.BlockSpec((_NB, D), lambda i: (i, 0)),
            pl.BlockSpec((_NB, 128), lambda i: (i, 0)),
            pl.BlockSpec((D, D), lambda i: (0, 0)),
            pl.BlockSpec((1, D), lambda i: (0, 0)),
            pl.BlockSpec((1, 1), lambda i: (0, 0)),
        ],
        out_specs=pl.BlockSpec((_NB, 1), lambda i: (i, 0)),
        out_shape=jax.ShapeDtypeStruct((N, 1), jnp.float32),
    )(x, agg, w2, wout2, bout2)


def _block_diag2(w):
    z = jnp.zeros((w.shape[0] * 2, w.shape[1] * 2), jnp.float32)
    z = z.at[: w.shape[0], : w.shape[1]].set(w)
    return z.at[w.shape[0]:, w.shape[1]:].set(w)


def kernel(bond_dist, params, atom_types, edge_index):
    pad = E_PAD - E
    fill = (jnp.arange(pad, dtype=jnp.int32) * 997) % N
    src = jnp.concatenate([edge_index[0].astype(jnp.int32), fill])
    dst = jnp.concatenate([edge_index[1].astype(jnp.int32), fill])
    src2d = src.reshape(E_PAD // 128, 128)
    dst2d = dst.reshape(E_PAD // 128, 128)
    bdp = jnp.concatenate(
        [bond_dist, jnp.full((pad,), 0.5, jnp.float32)]
    ).reshape(E2, 2)
    bde3 = bdp[:, 0].reshape(E2 // _PB, 1, _PB)
    bdo3 = bdp[:, 1].reshape(E2 // _PB, 1, _PB)
    t3 = atom_types.astype(jnp.int32).reshape(N // _NB, 1, _NB)

    zeros_half = jnp.zeros((N, D // 2), jnp.float32)
    wbw2 = _block_diag2(params["W_bw"])
    wbe2 = _block_diag2(params["W_be"])
    bbe2 = jnp.concatenate([params["b_be"], params["b_be"]]).reshape(1, 128)
    wout2 = params["W_out"].reshape(1, D)
    bout2 = params["b_out"].reshape(1, 1).astype(jnp.float32)

    x = _emb_tc(t3, params["atom_emb"])
    for k, blk in enumerate(params["blocks"]):
        g = _gather_sc(x, src2d)
        g128 = g.reshape(E2, 128)
        w1d = _block_diag2(blk["W1"])
        b1d = jnp.concatenate([blk["b1"], blk["b1"]]).reshape(1, 128)
        m_pair = _msg_tc(bde3, bdo3, g128, wbw2, wbe2, bbe2, w1d, b1d)
        m64 = m_pair.reshape(E_PAD, D)
        agg = _scatter_sc(m64, dst2d, zeros_half)
        if k < NBLOCKS - 1:
            x = _upd_tc(x, agg, blk["W2"])
        else:
            out = _final_tc(x, agg, blk["W2"], wout2, bout2)
    return out.reshape(N)


# speedup vs baseline: 4.5762x; 1.0605x over previous
import functools

import jax
import jax.numpy as jnp
from jax import lax
from jax.experimental import pallas as pl
from jax.experimental.pallas import tpu as pltpu
from jax.experimental.pallas import tpu_sc as plsc

N = 50000
E = 800000
D = 64
MAX_N = 9
CUTOFF = 5.0
NBLOCKS = 4
NELEM = 89

E_PAD = 819200
EH = E_PAD // 2
ECHUNK = 256
NSUB = ECHUNK // 128
NCORES = 2
NTILES = 16
NWORK = NCORES * NTILES
GCHUNK = 640
GSUB = GCHUNK // 128
FLUSH = 400
NFLUSH = N // FLUSH

_mesh = plsc.VectorSubcoreMesh(core_axis_name="c", subcore_axis_name="s")


def _make_gather(ne):
    chunks = ne // GCHUNK // NWORK

    @functools.partial(
        pl.kernel,
        out_type=jax.ShapeDtypeStruct((ne, D), jnp.float32),
        mesh=_mesh,
        scratch_types=[
            pltpu.VMEM((GSUB, 128), jnp.int32),
            pltpu.VMEM((GSUB, 128), jnp.int32),
            pltpu.VMEM((GCHUNK, D), jnp.float32),
            pltpu.VMEM((GCHUNK, D), jnp.float32),
            pltpu.SemaphoreType.DMA,
            pltpu.SemaphoreType.DMA,
            pltpu.SemaphoreType.DMA,
            pltpu.SemaphoreType.DMA,
            pltpu.SemaphoreType.DMA,
        ],
        compiler_params=pltpu.CompilerParams(use_tc_tiling_on_sc=False),
    )
    def gather_k(x_hbm, src2d_hbm, g_hbm, idx_v0, idx_v1, rows_v0, rows_v1,
                 sem_i0, sem_i1, sem_w0, sem_w1, sem_g):
        cid = lax.axis_index("c")
        sid = lax.axis_index("s")
        wid = sid * NCORES + cid
        base_chunk = wid * chunks
        idx_v = (idx_v0, idx_v1)
        rows_v = (rows_v0, rows_v1)
        sem_i = (sem_i0, sem_i1)
        sem_w = (sem_w0, sem_w1)

        def _fetch_idx(q, b):
            pltpu.async_copy(src2d_hbm.at[pl.ds(q * GSUB, GSUB)], idx_v[b],
                             sem_i[b])

        def _process(q, b, first):
            pltpu.make_async_copy(
                src2d_hbm.at[pl.ds(q * GSUB, GSUB)], idx_v[b], sem_i[b]
            ).wait()
            if not first:
                pltpu.make_async_copy(
                    rows_v[b], g_hbm.at[pl.ds(q * GCHUNK, GCHUNK)], sem_w[b]
                ).wait()
            descs = []
            for j in range(GSUB):
                descs.append(pltpu.async_copy(
                    x_hbm.at[idx_v[b].at[j]],
                    rows_v[b].at[pl.ds(j * 128, 128)],
                    sem_g,
                ))
            for d in descs:
                d.wait()
            pltpu.async_copy(rows_v[b], g_hbm.at[pl.ds(q * GCHUNK, GCHUNK)],
                             sem_w[b])

            @pl.when(q + 2 < base_chunk + chunks)
            def _():
                _fetch_idx(q + 2, b)

        for b in range(2):
            _fetch_idx(base_chunk + b, b)
        for b in range(2):
            _process(base_chunk + b, b, True)

        def body(i2, _):
            for b in range(2):
                _process(base_chunk + 2 + 2 * i2 + b, b, False)
            return 0

        lax.fori_loop(0, (chunks - 2) // 2, body, 0)
        for b in range(2):
            pltpu.make_async_copy(
                rows_v[b], g_hbm.at[pl.ds(base_chunk * GCHUNK, GCHUNK)],
                sem_w[b],
            ).wait()

    return gather_k


def _make_scatter(ne):
    chunks = ne // ECHUNK // NTILES

    @functools.partial(
        pl.kernel,
        out_type=jax.ShapeDtypeStruct((N, 128), jnp.float32),
        mesh=_mesh,
        scratch_types=[
            pltpu.VMEM_SHARED((N, D // 2), jnp.float32),
            pltpu.VMEM((NSUB, 128), jnp.int32),
            pltpu.VMEM((NSUB, 128), jnp.int32),
            pltpu.VMEM((ECHUNK, D // 2), jnp.float32),
            pltpu.VMEM((ECHUNK, D // 2), jnp.float32),
            pltpu.SemaphoreType.DMA,
            pltpu.SemaphoreType.DMA,
            pltpu.SemaphoreType.DMA,
            pltpu.SemaphoreType.DMA,
            pltpu.SemaphoreType.DMA,
        ],
        compiler_params=pltpu.CompilerParams(use_tc_tiling_on_sc=False),
    )
    def scatter_k(m_hbm, dst2d_hbm, zeros_hbm, agg_hbm, acc_sh,
                  idx_v0, idx_v1, mrow_v0, mrow_v1,
                  sem_i0, sem_i1, sem_m0, sem_m1, sem_s):
        cid = lax.axis_index("c")
        sid = lax.axis_index("s")
        c0 = cid * (D // 2)
        idx_v = (idx_v0, idx_v1)
        mrow_v = (mrow_v0, mrow_v1)
        sem_i = (sem_i0, sem_i1)
        sem_m = (sem_m0, sem_m1)

        def zero(k, _):
            q = sid + k * NTILES

            @pl.when(q < NFLUSH)
            def _():
                r0 = q * FLUSH
                pltpu.sync_copy(zeros_hbm.at[pl.ds(r0, FLUSH)],
                                acc_sh.at[pl.ds(r0, FLUSH)])
            return 0

        lax.fori_loop(0, (NFLUSH + NTILES - 1) // NTILES, zero, 0)
        plsc.subcore_barrier()

        base_chunk = sid * chunks

        def _fetch(q, b):
            pltpu.async_copy(dst2d_hbm.at[pl.ds(q * NSUB, NSUB)], idx_v[b],
                             sem_i[b])
            pltpu.async_copy(
                m_hbm.at[pl.ds(q * ECHUNK, ECHUNK), pl.ds(c0, D // 2)],
                mrow_v[b], sem_m[b])

        for b in range(2):
            _fetch(base_chunk + b, b)

        def body(i2, _):
            for b in range(2):
                i = 2 * i2 + b
                q = base_chunk + i
                pltpu.make_async_copy(
                    dst2d_hbm.at[pl.ds(q * NSUB, NSUB)], idx_v[b], sem_i[b]
                ).wait()
                pltpu.make_async_copy(
                    m_hbm.at[pl.ds(q * ECHUNK, ECHUNK), pl.ds(c0, D // 2)],
                    mrow_v[b], sem_m[b],
                ).wait()
                descs = []
                for j in range(NSUB):
                    descs.append(pltpu.async_copy(
                        mrow_v[b].at[pl.ds(j * 128, 128)],
                        acc_sh.at[idx_v[b].at[j]],
                        sem_s, add=True,
                    ))
                for d in descs:
                    d.wait()

                @pl.when(i + 2 < chunks)
                def _():
                    _fetch(q + 2, b)
            return 0

        lax.fori_loop(0, chunks // 2, body, 0)
        plsc.subcore_barrier()

        def flush(k, _):
            q = sid + k * NTILES

            @pl.when(q < NFLUSH)
            def _():
                r0 = q * FLUSH
                pltpu.sync_copy(
                    acc_sh.at[pl.ds(r0, FLUSH)],
                    agg_hbm.at[pl.ds(r0, FLUSH), pl.ds(c0, D // 2)])
            return 0

        lax.fori_loop(0, (NFLUSH + NTILES - 1) // NTILES, flush, 0)

    return scatter_k


_gather_sc = _make_gather(EH)
_scatter_sc = _make_scatter(EH)


_EB = 2048
_PB = _EB // 2
_NB = 1000


def _bessel_rows(bd):
    r = 0.5 + bd * (CUTOFF - 0.5)
    theta = r * (jnp.pi / CUTOFF)
    s1 = jnp.sin(theta)
    c1 = jnp.cos(theta)
    env = 0.5 * (c1 + 1.0)
    pref = jnp.sqrt(2.0 / CUTOFF) * env / r
    two_c1 = 2.0 * c1
    rows = [s1 * pref]
    s_prev, s_cur = jnp.zeros_like(s1), s1
    for _ in range(MAX_N - 1):
        s_prev, s_cur = s_cur, two_c1 * s_cur - s_prev
        rows.append(s_cur * pref)
    return rows


def _msg_body(row_off, bde_ref, bdo_ref, g_ref, wbw2_ref, wbe2_ref, bbe2_ref,
              w1d_ref, b1d_ref, m_ref):
    i = pl.program_id(0)
    rows = _bessel_rows(bde_ref[0, 0, :]) + _bessel_rows(bdo_ref[0, 0, :])
    smooth_t = jnp.concatenate([x[None, :] for x in rows], axis=0)
    cdims = (((0,), (0,)), ((), ()))
    bw = lax.dot_general(smooth_t, wbw2_ref[...], cdims,
                         preferred_element_type=jnp.float32)
    bf = jax.nn.silu(
        lax.dot_general(smooth_t, wbe2_ref[...], cdims,
                        preferred_element_type=jnp.float32)
        + bbe2_ref[...])
    h = g_ref[...] * bw + bf
    mm = jnp.dot(h, w1d_ref[...], preferred_element_type=jnp.float32) \
        + b1d_ref[...]
    m = jax.nn.silu(mm)
    rowid = row_off + i * _PB + lax.broadcasted_iota(jnp.int32, (_PB, 1), 0)
    m = jnp.where(rowid < E // 2, m, 0.0)
    m_ref[...] = m


def _msg_tc(row_off, bde3, bdo3, g128, wbw2, wbe2, bbe2, w1d, b1d):
    eh2 = EH // 2
    return pl.pallas_call(
        functools.partial(_msg_body, row_off),
        grid=(eh2 // _PB,),
        in_specs=[
            pl.BlockSpec((1, 1, _PB), lambda i: (i, 0, 0)),
            pl.BlockSpec((1, 1, _PB), lambda i: (i, 0, 0)),
            pl.BlockSpec((_PB, 128), lambda i: (i, 0)),
            pl.BlockSpec((2 * MAX_N, 128), lambda i: (0, 0)),
            pl.BlockSpec((2 * MAX_N, 128), lambda i: (0, 0)),
            pl.BlockSpec((1, 128), lambda i: (0, 0)),
            pl.BlockSpec((128, 128), lambda i: (0, 0)),
            pl.BlockSpec((1, 128), lambda i: (0, 0)),
        ],
        out_specs=pl.BlockSpec((_PB, 128), lambda i: (i, 0)),
        out_shape=jax.ShapeDtypeStruct((eh2, 128), jnp.float32),
    )(bde3, bdo3, g128, wbw2, wbe2, bbe2, w1d, b1d)


def _emb_body(t_ref, emb_ref, x_ref):
    t = t_ref[0, 0, :]
    oh = (t[:, None] == lax.broadcasted_iota(jnp.int32, (_NB, NELEM), 1)
          ).astype(jnp.float32)
    x_ref[...] = jnp.dot(oh, emb_ref[...], preferred_element_type=jnp.float32)


def _emb_tc(t3, emb):
    return pl.pallas_call(
        _emb_body,
        grid=(N // _NB,),
        in_specs=[
            pl.BlockSpec((1, 1, _NB), lambda i: (i, 0, 0)),
            pl.BlockSpec((NELEM, D), lambda i: (0, 0)),
        ],
        out_specs=pl.BlockSpec((_NB, D), lambda i: (i, 0)),
        out_shape=jax.ShapeDtypeStruct((N, D), jnp.float32),
    )(t3, emb)


def _upd_body(x_ref, agga_ref, aggb_ref, w2_ref, o_ref):
    a = agga_ref[:, :D] + aggb_ref[:, :D]
    y = jnp.dot(a, w2_ref[...], preferred_element_type=jnp.float32)
    o_ref[...] = x_ref[...] + y


def _upd_tc(x, agga, aggb, w2):
    return pl.pallas_call(
        _upd_body,
        grid=(N // _NB,),
        in_specs=[
            pl.BlockSpec((_NB, D), lambda i: (i, 0)),
            pl.BlockSpec((_NB, 128), lambda i: (i, 0)),
            pl.BlockSpec((_NB, 128), lambda i: (i, 0)),
            pl.BlockSpec((D, D), lambda i: (0, 0)),
        ],
        out_specs=pl.BlockSpec((_NB, D), lambda i: (i, 0)),
        out_shape=jax.ShapeDtypeStruct((N, D), jnp.float32),
    )(x, agga, aggb, w2)


def _final_body(x_ref, agga_ref, aggb_ref, w2_ref, wout_ref, bout_ref, o_ref):
    a = agga_ref[:, :D] + aggb_ref[:, :D]
    y = jnp.dot(a, w2_ref[...], preferred_element_type=jnp.float32)
    xf = x_ref[...] + y
    e = jnp.sum(xf * wout_ref[...], axis=1, keepdims=True) + bout_ref[0, 0]
    o_ref[...] = e


def _final_tc(x, agga, aggb, w2, wout2, bout2):
    return pl.pallas_call(
        _final_body,
        grid=(N // _NB,),
        in_specs=[
            pl.BlockSpec((_NB, D), lambda i: (i, 0)),
            pl.BlockSpec((_NB, 128), lambda i: (i, 0)),
            pl.BlockSpec((_NB, 128), lambda i: (i, 0)),
            pl.BlockSpec((D, D), lambda i: (0, 0)),
            pl.BlockSpec((1, D), lambda i: (0, 0)),
            pl.BlockSpec((1, 1), lambda i: (0, 0)),
        ],
        out_specs=pl.BlockSpec((_NB, 1), lambda i: (i, 0)),
        out_shape=jax.ShapeDtypeStruct((N, 1), jnp.float32),
    )(x, agga, aggb, w2, wout2, bout2)


def _block_diag2(w):
    z = jnp.zeros((w.shape[0] * 2, w.shape[1] * 2), jnp.float32)
    z = z.at[: w.shape[0], : w.shape[1]].set(w)
    return z.at[w.shape[0]:, w.shape[1]:].set(w)


def kernel(bond_dist, params, atom_types, edge_index):
    pad = E_PAD - E
    fill = (jnp.arange(pad, dtype=jnp.int32) * 997) % N
    src = jnp.concatenate([edge_index[0].astype(jnp.int32), fill])
    dst = jnp.concatenate([edge_index[1].astype(jnp.int32), fill])
    src2d = [src[:EH].reshape(EH // 128, 128), src[EH:].reshape(EH // 128, 128)]
    dst2d = [dst[:EH].reshape(EH // 128, 128), dst[EH:].reshape(EH // 128, 128)]
    bd_pad = jnp.concatenate([bond_dist, jnp.full((pad,), 0.5, jnp.float32)])
    eh2 = EH // 2
    bde3, bdo3 = [], []
    for h in range(2):
        bdp = bd_pad[h * EH:(h + 1) * EH].reshape(eh2, 2)
        bde3.append(bdp[:, 0].reshape(eh2 // _PB, 1, _PB))
        bdo3.append(bdp[:, 1].reshape(eh2 // _PB, 1, _PB))
    t3 = atom_types.astype(jnp.int32).reshape(N // _NB, 1, _NB)

    zeros_half = jnp.zeros((N, D // 2), jnp.float32)
    wbw2 = _block_diag2(params["W_bw"])
    wbe2 = _block_diag2(params["W_be"])
    bbe2 = jnp.concatenate([params["b_be"], params["b_be"]]).reshape(1, 128)
    wout2 = params["W_out"].reshape(1, D)
    bout2 = params["b_out"].reshape(1, 1).astype(jnp.float32)

    x = _emb_tc(t3, params["atom_emb"])
    row_offs = [0, eh2]
    for k, blk in enumerate(params["blocks"]):
        w1d = _block_diag2(blk["W1"])
        b1d = jnp.concatenate([blk["b1"], blk["b1"]]).reshape(1, 128)
        g = [_gather_sc(x, src2d[h]) for h in range(2)]
        m = [
            _msg_tc(row_offs[h], bde3[h], bdo3[h], g[h].reshape(eh2, 128),
                    wbw2, wbe2, bbe2, w1d, b1d)
            for h in range(2)
        ]
        agg = [_scatter_sc(m[h].reshape(EH, D), dst2d[h], zeros_half)
               for h in range(2)]
        if k < NBLOCKS - 1:
            x = _upd_tc(x, agg[0], agg[1], blk["W2"])
        else:
            out = _final_tc(x, agg[0], agg[1], blk["W2"], wout2, bout2)
    return out.reshape(N)


# speedup vs baseline: 5.3383x; 1.1665x over previous
import functools

import jax
import jax.numpy as jnp
from jax import lax
from jax.experimental import pallas as pl
from jax.experimental.pallas import tpu as pltpu
from jax.experimental.pallas import tpu_sc as plsc

N = 50000
E = 800000
D = 64
MAX_N = 9
CUTOFF = 5.0
NBLOCKS = 4
NELEM = 89

E_PAD = 819200
EH = E_PAD // 2
ECHUNK = 256
NSUB = ECHUNK // 128
NCORES = 2
NTILES = 16
NWORK = NCORES * NTILES
GCHUNK = 640
GSUB = GCHUNK // 128
FLUSH = 400
NFLUSH = N // FLUSH

_mesh = plsc.VectorSubcoreMesh(core_axis_name="c", subcore_axis_name="s")


def _make_gather(ne):
    chunks = ne // GCHUNK // NWORK

    @functools.partial(
        pl.kernel,
        out_type=jax.ShapeDtypeStruct((ne, D), jnp.float32),
        mesh=_mesh,
        scratch_types=[
            pltpu.VMEM((GSUB, 128), jnp.int32),
            pltpu.VMEM((GSUB, 128), jnp.int32),
            pltpu.VMEM((GCHUNK, D), jnp.float32),
            pltpu.VMEM((GCHUNK, D), jnp.float32),
            pltpu.SemaphoreType.DMA,
            pltpu.SemaphoreType.DMA,
            pltpu.SemaphoreType.DMA,
            pltpu.SemaphoreType.DMA,
            pltpu.SemaphoreType.DMA,
        ],
        compiler_params=pltpu.CompilerParams(use_tc_tiling_on_sc=False),
    )
    def gather_k(x_hbm, src2d_hbm, g_hbm, idx_v0, idx_v1, rows_v0, rows_v1,
                 sem_i0, sem_i1, sem_w0, sem_w1, sem_g):
        cid = lax.axis_index("c")
        sid = lax.axis_index("s")
        wid = sid * NCORES + cid
        base_chunk = wid * chunks
        idx_v = (idx_v0, idx_v1)
        rows_v = (rows_v0, rows_v1)
        sem_i = (sem_i0, sem_i1)
        sem_w = (sem_w0, sem_w1)

        def _fetch_idx(q, b):
            pltpu.async_copy(src2d_hbm.at[pl.ds(q * GSUB, GSUB)], idx_v[b],
                             sem_i[b])

        def _process(q, b, first):
            pltpu.make_async_copy(
                src2d_hbm.at[pl.ds(q * GSUB, GSUB)], idx_v[b], sem_i[b]
            ).wait()
            if not first:
                pltpu.make_async_copy(
                    rows_v[b], g_hbm.at[pl.ds(q * GCHUNK, GCHUNK)], sem_w[b]
                ).wait()
            descs = []
            for j in range(GSUB):
                descs.append(pltpu.async_copy(
                    x_hbm.at[idx_v[b].at[j]],
                    rows_v[b].at[pl.ds(j * 128, 128)],
                    sem_g,
                ))
            for d in descs:
                d.wait()
            pltpu.async_copy(rows_v[b], g_hbm.at[pl.ds(q * GCHUNK, GCHUNK)],
                             sem_w[b])

            @pl.when(q + 2 < base_chunk + chunks)
            def _():
                _fetch_idx(q + 2, b)

        for b in range(2):
            _fetch_idx(base_chunk + b, b)
        for b in range(2):
            _process(base_chunk + b, b, True)

        def body(i2, _):
            for b in range(2):
                _process(base_chunk + 2 + 2 * i2 + b, b, False)
            return 0

        lax.fori_loop(0, (chunks - 2) // 2, body, 0)
        for b in range(2):
            pltpu.make_async_copy(
                rows_v[b], g_hbm.at[pl.ds(base_chunk * GCHUNK, GCHUNK)],
                sem_w[b],
            ).wait()

    return gather_k


def _make_scatter(ne):
    chunks = ne // ECHUNK // NTILES

    @functools.partial(
        pl.kernel,
        out_type=jax.ShapeDtypeStruct((N, 128), jnp.float32),
        mesh=_mesh,
        scratch_types=[
            pltpu.VMEM_SHARED((N, D // 2), jnp.float32),
            pltpu.VMEM((NSUB, 128), jnp.int32),
            pltpu.VMEM((NSUB, 128), jnp.int32),
            pltpu.VMEM((ECHUNK, D // 2), jnp.float32),
            pltpu.VMEM((ECHUNK, D // 2), jnp.float32),
            pltpu.SemaphoreType.DMA,
            pltpu.SemaphoreType.DMA,
            pltpu.SemaphoreType.DMA,
            pltpu.SemaphoreType.DMA,
            pltpu.SemaphoreType.DMA,
        ],
        compiler_params=pltpu.CompilerParams(use_tc_tiling_on_sc=False),
    )
    def scatter_k(m_hbm, dst2d_hbm, zeros_hbm, agg_hbm, acc_sh,
                  idx_v0, idx_v1, mrow_v0, mrow_v1,
                  sem_i0, sem_i1, sem_m0, sem_m1, sem_s):
        cid = lax.axis_index("c")
        sid = lax.axis_index("s")
        c0 = cid * (D // 2)
        idx_v = (idx_v0, idx_v1)
        mrow_v = (mrow_v0, mrow_v1)
        sem_i = (sem_i0, sem_i1)
        sem_m = (sem_m0, sem_m1)

        def zero(k, _):
            q = sid + k * NTILES

            @pl.when(q < NFLUSH)
            def _():
                r0 = q * FLUSH
                pltpu.sync_copy(zeros_hbm.at[pl.ds(r0, FLUSH)],
                                acc_sh.at[pl.ds(r0, FLUSH)])
            return 0

        lax.fori_loop(0, (NFLUSH + NTILES - 1) // NTILES, zero, 0)
        plsc.subcore_barrier()

        base_chunk = sid * chunks

        def _fetch(q, b):
            pltpu.async_copy(dst2d_hbm.at[pl.ds(q * NSUB, NSUB)], idx_v[b],
                             sem_i[b])
            pltpu.async_copy(
                m_hbm.at[pl.ds(q * ECHUNK, ECHUNK), pl.ds(c0, D // 2)],
                mrow_v[b], sem_m[b])

        for b in range(2):
            _fetch(base_chunk + b, b)

        def body(i2, _):
            for b in range(2):
                i = 2 * i2 + b
                q = base_chunk + i
                pltpu.make_async_copy(
                    dst2d_hbm.at[pl.ds(q * NSUB, NSUB)], idx_v[b], sem_i[b]
                ).wait()
                pltpu.make_async_copy(
                    m_hbm.at[pl.ds(q * ECHUNK, ECHUNK), pl.ds(c0, D // 2)],
                    mrow_v[b], sem_m[b],
                ).wait()
                descs = []
                for j in range(NSUB):
                    descs.append(pltpu.async_copy(
                        mrow_v[b].at[pl.ds(j * 128, 128)],
                        acc_sh.at[idx_v[b].at[j]],
                        sem_s, add=True,
                    ))
                for d in descs:
                    d.wait()

                @pl.when(i + 2 < chunks)
                def _():
                    _fetch(q + 2, b)
            return 0

        lax.fori_loop(0, chunks // 2, body, 0)
        plsc.subcore_barrier()

        def flush(k, _):
            q = sid + k * NTILES

            @pl.when(q < NFLUSH)
            def _():
                r0 = q * FLUSH
                pltpu.sync_copy(
                    acc_sh.at[pl.ds(r0, FLUSH)],
                    agg_hbm.at[pl.ds(r0, FLUSH), pl.ds(c0, D // 2)])
            return 0

        lax.fori_loop(0, (NFLUSH + NTILES - 1) // NTILES, flush, 0)

    return scatter_k


_gather_sc = _make_gather(EH)
_scatter_sc = _make_scatter(EH)


_EB = 4096
_PB = _EB // 2
_NB = 1000


def _bessel_rows(bd):
    r = 0.5 + bd * (CUTOFF - 0.5)
    theta = r * (jnp.pi / CUTOFF)
    s1 = jnp.sin(theta)
    c1 = jnp.cos(theta)
    env = 0.5 * (c1 + 1.0)
    pref = jnp.sqrt(2.0 / CUTOFF) * env / r
    two_c1 = 2.0 * c1
    rows = [s1 * pref]
    s_prev, s_cur = jnp.zeros_like(s1), s1
    for _ in range(MAX_N - 1):
        s_prev, s_cur = s_cur, two_c1 * s_cur - s_prev
        rows.append(s_cur * pref)
    return rows


def _msg_body(row_off, bde_ref, bdo_ref, g_ref, wbw2_ref, wbe2_ref, bbe2_ref,
              w1d_ref, b1d_ref, m_ref):
    i = pl.program_id(0)
    rows = _bessel_rows(bde_ref[0, 0, :]) + _bessel_rows(bdo_ref[0, 0, :])
    smooth_t = jnp.concatenate([x[None, :] for x in rows], axis=0)
    cdims = (((0,), (0,)), ((), ()))
    bw = lax.dot_general(smooth_t, wbw2_ref[...], cdims,
                         preferred_element_type=jnp.float32)
    bf = jax.nn.silu(
        lax.dot_general(smooth_t, wbe2_ref[...], cdims,
                        preferred_element_type=jnp.float32)
        + bbe2_ref[...])
    h = g_ref[...] * bw + bf
    mm = jnp.dot(h, w1d_ref[...], preferred_element_type=jnp.float32) \
        + b1d_ref[...]
    m = jax.nn.silu(mm)
    rowid = row_off + i * _PB + lax.broadcasted_iota(jnp.int32, (_PB, 1), 0)
    m = jnp.where(rowid < E // 2, m, 0.0)
    m_ref[...] = m


def _msg_tc(row_off, bde3, bdo3, g128, wbw2, wbe2, bbe2, w1d, b1d):
    eh2 = EH // 2
    return pl.pallas_call(
        functools.partial(_msg_body, row_off),
        grid=(eh2 // _PB,),
        in_specs=[
            pl.BlockSpec((1, 1, _PB), lambda i: (i, 0, 0)),
            pl.BlockSpec((1, 1, _PB), lambda i: (i, 0, 0)),
            pl.BlockSpec((_PB, 128), lambda i: (i, 0)),
            pl.BlockSpec((2 * MAX_N, 128), lambda i: (0, 0)),
            pl.BlockSpec((2 * MAX_N, 128), lambda i: (0, 0)),
            pl.BlockSpec((1, 128), lambda i: (0, 0)),
            pl.BlockSpec((128, 128), lambda i: (0, 0)),
            pl.BlockSpec((1, 128), lambda i: (0, 0)),
        ],
        out_specs=pl.BlockSpec((_PB, 128), lambda i: (i, 0)),
        out_shape=jax.ShapeDtypeStruct((eh2, 128), jnp.float32),
    )(bde3, bdo3, g128, wbw2, wbe2, bbe2, w1d, b1d)


def _emb_body(t_ref, emb_ref, x_ref):
    t = t_ref[0, 0, :]
    oh = (t[:, None] == lax.broadcasted_iota(jnp.int32, (_NB, NELEM), 1)
          ).astype(jnp.float32)
    x_ref[...] = jnp.dot(oh, emb_ref[...], preferred_element_type=jnp.float32)


def _emb_tc(t3, emb):
    return pl.pallas_call(
        _emb_body,
        grid=(N // _NB,),
        in_specs=[
            pl.BlockSpec((1, 1, _NB), lambda i: (i, 0, 0)),
            pl.BlockSpec((NELEM, D), lambda i: (0, 0)),
        ],
        out_specs=pl.BlockSpec((_NB, D), lambda i: (i, 0)),
        out_shape=jax.ShapeDtypeStruct((N, D), jnp.float32),
    )(t3, emb)


def _upd_body(x_ref, agga_ref, aggb_ref, w2_ref, o_ref):
    a = agga_ref[:, :D] + aggb_ref[:, :D]
    y = jnp.dot(a, w2_ref[...], preferred_element_type=jnp.float32)
    o_ref[...] = x_ref[...] + y


def _upd_tc(x, agga, aggb, w2):
    return pl.pallas_call(
        _upd_body,
        grid=(N // _NB,),
        in_specs=[
            pl.BlockSpec((_NB, D), lambda i: (i, 0)),
            pl.BlockSpec((_NB, 128), lambda i: (i, 0)),
            pl.BlockSpec((_NB, 128), lambda i: (i, 0)),
            pl.BlockSpec((D, D), lambda i: (0, 0)),
        ],
        out_specs=pl.BlockSpec((_NB, D), lambda i: (i, 0)),
        out_shape=jax.ShapeDtypeStruct((N, D), jnp.float32),
    )(x, agga, aggb, w2)


def _final_body(x_ref, agga_ref, aggb_ref, w2_ref, wout_ref, bout_ref, o_ref):
    a = agga_ref[:, :D] + aggb_ref[:, :D]
    y = jnp.dot(a, w2_ref[...], preferred_element_type=jnp.float32)
    xf = x_ref[...] + y
    e = jnp.sum(xf * wout_ref[...], axis=1, keepdims=True) + bout_ref[0, 0]
    o_ref[...] = e


def _final_tc(x, agga, aggb, w2, wout2, bout2):
    return pl.pallas_call(
        _final_body,
        grid=(N // _NB,),
        in_specs=[
            pl.BlockSpec((_NB, D), lambda i: (i, 0)),
            pl.BlockSpec((_NB, 128), lambda i: (i, 0)),
            pl.BlockSpec((_NB, 128), lambda i: (i, 0)),
            pl.BlockSpec((D, D), lambda i: (0, 0)),
            pl.BlockSpec((1, D), lambda i: (0, 0)),
            pl.BlockSpec((1, 1), lambda i: (0, 0)),
        ],
        out_specs=pl.BlockSpec((_NB, 1), lambda i: (i, 0)),
        out_shape=jax.ShapeDtypeStruct((N, 1), jnp.float32),
    )(x, agga, aggb, w2, wout2, bout2)


def _block_diag2(w):
    z = jnp.zeros((w.shape[0] * 2, w.shape[1] * 2), jnp.float32)
    z = z.at[: w.shape[0], : w.shape[1]].set(w)
    return z.at[w.shape[0]:, w.shape[1]:].set(w)


def kernel(bond_dist, params, atom_types, edge_index):
    pad = E_PAD - E
    fill = (jnp.arange(pad, dtype=jnp.int32) * 997) % N
    src = jnp.concatenate([edge_index[0].astype(jnp.int32), fill])
    dst = jnp.concatenate([edge_index[1].astype(jnp.int32), fill])
    src2d = [src[:EH].reshape(EH // 128, 128), src[EH:].reshape(EH // 128, 128)]
    dst2d = [dst[:EH].reshape(EH // 128, 128), dst[EH:].reshape(EH // 128, 128)]
    bd_pad = jnp.concatenate([bond_dist, jnp.full((pad,), 0.5, jnp.float32)])
    eh2 = EH // 2
    bde3, bdo3 = [], []
    for h in range(2):
        bdp = bd_pad[h * EH:(h + 1) * EH].reshape(eh2, 2)
        bde3.append(bdp[:, 0].reshape(eh2 // _PB, 1, _PB))
        bdo3.append(bdp[:, 1].reshape(eh2 // _PB, 1, _PB))
    t3 = atom_types.astype(jnp.int32).reshape(N // _NB, 1, _NB)

    zeros_half = jnp.zeros((N, D // 2), jnp.float32)
    wbw2 = _block_diag2(params["W_bw"])
    wbe2 = _block_diag2(params["W_be"])
    bbe2 = jnp.concatenate([params["b_be"], params["b_be"]]).reshape(1, 128)
    wout2 = params["W_out"].reshape(1, D)
    bout2 = params["b_out"].reshape(1, 1).astype(jnp.float32)

    x = _emb_tc(t3, params["atom_emb"])
    row_offs = [0, eh2]
    for k, blk in enumerate(params["blocks"]):
        w1d = _block_diag2(blk["W1"])
        b1d = jnp.concatenate([blk["b1"], blk["b1"]]).reshape(1, 128)
        g = [_gather_sc(x, src2d[h]) for h in range(2)]
        m = [
            _msg_tc(row_offs[h], bde3[h], bdo3[h], g[h].reshape(eh2, 128),
                    wbw2, wbe2, bbe2, w1d, b1d)
            for h in range(2)
        ]
        agg = [_scatter_sc(m[h].reshape(EH, D), dst2d[h], zeros_half)
               for h in range(2)]
        if k < NBLOCKS - 1:
            x = _upd_tc(x, agg[0], agg[1], blk["W2"])
        else:
            out = _final_tc(x, agg[0], agg[1], blk["W2"], wout2, bout2)
    return out.reshape(N)


# speedup vs baseline: 5.6833x; 1.0646x over previous
import functools

import jax
import jax.numpy as jnp
from jax import lax
from jax.experimental import pallas as pl
from jax.experimental.pallas import tpu as pltpu
from jax.experimental.pallas import tpu_sc as plsc

N = 50000
E = 800000
D = 64
MAX_N = 9
CUTOFF = 5.0
NBLOCKS = 4
NELEM = 89

E_PAD = 819200
EH = E_PAD // 2
ECHUNK = 256
NSUB = ECHUNK // 128
NCORES = 2
NTILES = 16
NWORK = NCORES * NTILES
GCHUNK = 640
GSUB = GCHUNK // 128
FLUSH = 400
NFLUSH = N // FLUSH

_mesh = plsc.VectorSubcoreMesh(core_axis_name="c", subcore_axis_name="s")


def _make_gather(ne):
    chunks = ne // GCHUNK // NWORK

    @functools.partial(
        pl.kernel,
        out_type=jax.ShapeDtypeStruct((ne, D), jnp.float32),
        mesh=_mesh,
        scratch_types=[
            pltpu.VMEM((GSUB, 128), jnp.int32),
            pltpu.VMEM((GSUB, 128), jnp.int32),
            pltpu.VMEM((GCHUNK, D), jnp.float32),
            pltpu.VMEM((GCHUNK, D), jnp.float32),
            pltpu.SemaphoreType.DMA,
            pltpu.SemaphoreType.DMA,
            pltpu.SemaphoreType.DMA,
            pltpu.SemaphoreType.DMA,
            pltpu.SemaphoreType.DMA,
        ],
        compiler_params=pltpu.CompilerParams(use_tc_tiling_on_sc=False),
    )
    def gather_k(x_hbm, src2d_hbm, g_hbm, idx_v0, idx_v1, rows_v0, rows_v1,
                 sem_i0, sem_i1, sem_w0, sem_w1, sem_g):
        cid = lax.axis_index("c")
        sid = lax.axis_index("s")
        wid = sid * NCORES + cid
        base_chunk = wid * chunks
        idx_v = (idx_v0, idx_v1)
        rows_v = (rows_v0, rows_v1)
        sem_i = (sem_i0, sem_i1)
        sem_w = (sem_w0, sem_w1)

        def _fetch_idx(q, b):
            pltpu.async_copy(src2d_hbm.at[pl.ds(q * GSUB, GSUB)], idx_v[b],
                             sem_i[b])

        def _process(q, b, first):
            pltpu.make_async_copy(
                src2d_hbm.at[pl.ds(q * GSUB, GSUB)], idx_v[b], sem_i[b]
            ).wait()
            if not first:
                pltpu.make_async_copy(
                    rows_v[b], g_hbm.at[pl.ds(q * GCHUNK, GCHUNK)], sem_w[b]
                ).wait()
            descs = []
            for j in range(GSUB):
                descs.append(pltpu.async_copy(
                    x_hbm.at[idx_v[b].at[j]],
                    rows_v[b].at[pl.ds(j * 128, 128)],
                    sem_g,
                ))
            for d in descs:
                d.wait()
            pltpu.async_copy(rows_v[b], g_hbm.at[pl.ds(q * GCHUNK, GCHUNK)],
                             sem_w[b])

            @pl.when(q + 2 < base_chunk + chunks)
            def _():
                _fetch_idx(q + 2, b)

        for b in range(2):
            _fetch_idx(base_chunk + b, b)
        for b in range(2):
            _process(base_chunk + b, b, True)

        def body(i2, _):
            for b in range(2):
                _process(base_chunk + 2 + 2 * i2 + b, b, False)
            return 0

        lax.fori_loop(0, (chunks - 2) // 2, body, 0)
        for b in range(2):
            pltpu.make_async_copy(
                rows_v[b], g_hbm.at[pl.ds(base_chunk * GCHUNK, GCHUNK)],
                sem_w[b],
            ).wait()

    return gather_k


def _make_scatter(ne):
    chunks = ne // ECHUNK // NTILES

    @functools.partial(
        pl.kernel,
        out_type=jax.ShapeDtypeStruct((N, 128), jnp.float32),
        mesh=_mesh,
        scratch_types=[
            pltpu.VMEM_SHARED((N, D // 2), jnp.float32),
            pltpu.VMEM((NSUB, 128), jnp.int32),
            pltpu.VMEM((NSUB, 128), jnp.int32),
            pltpu.VMEM((ECHUNK, D // 2), jnp.float32),
            pltpu.VMEM((ECHUNK, D // 2), jnp.float32),
            pltpu.SemaphoreType.DMA,
            pltpu.SemaphoreType.DMA,
            pltpu.SemaphoreType.DMA,
            pltpu.SemaphoreType.DMA,
            pltpu.SemaphoreType.DMA,
        ],
        compiler_params=pltpu.CompilerParams(use_tc_tiling_on_sc=False),
    )
    def scatter_k(m_hbm, dst2d_hbm, zeros_hbm, agg_hbm, acc_sh,
                  idx_v0, idx_v1, mrow_v0, mrow_v1,
                  sem_i0, sem_i1, sem_m0, sem_m1, sem_s):
        cid = lax.axis_index("c")
        sid = lax.axis_index("s")
        c0 = cid * (D // 2)
        idx_v = (idx_v0, idx_v1)
        mrow_v = (mrow_v0, mrow_v1)
        sem_i = (sem_i0, sem_i1)
        sem_m = (sem_m0, sem_m1)

        def zero(k, _):
            q = sid + k * NTILES

            @pl.when(q < NFLUSH)
            def _():
                r0 = q * FLUSH
                pltpu.sync_copy(zeros_hbm.at[pl.ds(r0, FLUSH)],
                                acc_sh.at[pl.ds(r0, FLUSH)])
            return 0

        lax.fori_loop(0, (NFLUSH + NTILES - 1) // NTILES, zero, 0)
        plsc.subcore_barrier()

        base_chunk = sid * chunks

        def _fetch(q, b):
            pltpu.async_copy(dst2d_hbm.at[pl.ds(q * NSUB, NSUB)], idx_v[b],
                             sem_i[b])
            pltpu.async_copy(
                m_hbm.at[pl.ds(q * ECHUNK, ECHUNK), pl.ds(c0, D // 2)],
                mrow_v[b], sem_m[b])

        for b in range(2):
            _fetch(base_chunk + b, b)

        def body(i2, _):
            for b in range(2):
                i = 2 * i2 + b
                q = base_chunk + i
                pltpu.make_async_copy(
                    dst2d_hbm.at[pl.ds(q * NSUB, NSUB)], idx_v[b], sem_i[b]
                ).wait()
                pltpu.make_async_copy(
                    m_hbm.at[pl.ds(q * ECHUNK, ECHUNK), pl.ds(c0, D // 2)],
                    mrow_v[b], sem_m[b],
                ).wait()
                descs = []
                for j in range(NSUB):
                    descs.append(pltpu.async_copy(
                        mrow_v[b].at[pl.ds(j * 128, 128)],
                        acc_sh.at[idx_v[b].at[j]],
                        sem_s, add=True,
                    ))
                for d in descs:
                    d.wait()

                @pl.when(i + 2 < chunks)
                def _():
                    _fetch(q + 2, b)
            return 0

        lax.fori_loop(0, chunks // 2, body, 0)
        plsc.subcore_barrier()

        def flush(k, _):
            q = sid + k * NTILES

            @pl.when(q < NFLUSH)
            def _():
                r0 = q * FLUSH
                pltpu.sync_copy(
                    acc_sh.at[pl.ds(r0, FLUSH)],
                    agg_hbm.at[pl.ds(r0, FLUSH), pl.ds(c0, D // 2)])
            return 0

        lax.fori_loop(0, (NFLUSH + NTILES - 1) // NTILES, flush, 0)

    return scatter_k


_gather_sc = _make_gather(EH)
_scatter_sc = _make_scatter(EH)


_EB = 8192
_PB = _EB // 2
_NB = 1000


def _bessel_rows(bd):
    r = 0.5 + bd * (CUTOFF - 0.5)
    theta = r * (jnp.pi / CUTOFF)
    s1 = jnp.sin(theta)
    c1 = jnp.cos(theta)
    env = 0.5 * (c1 + 1.0)
    pref = jnp.sqrt(2.0 / CUTOFF) * env / r
    two_c1 = 2.0 * c1
    rows = [s1 * pref]
    s_prev, s_cur = jnp.zeros_like(s1), s1
    for _ in range(MAX_N - 1):
        s_prev, s_cur = s_cur, two_c1 * s_cur - s_prev
        rows.append(s_cur * pref)
    return rows


def _msg_body(row_off, bde_ref, bdo_ref, g_ref, wbw2_ref, wbe2_ref, bbe2_ref,
              w1d_ref, b1d_ref, m_ref):
    i = pl.program_id(0)
    rows = _bessel_rows(bde_ref[0, 0, :]) + _bessel_rows(bdo_ref[0, 0, :])
    smooth_t = jnp.concatenate([x[None, :] for x in rows], axis=0)
    cdims = (((0,), (0,)), ((), ()))
    bw = lax.dot_general(smooth_t, wbw2_ref[...], cdims,
                         preferred_element_type=jnp.float32)
    bf = jax.nn.silu(
        lax.dot_general(smooth_t, wbe2_ref[...], cdims,
                        preferred_element_type=jnp.float32)
        + bbe2_ref[...])
    h = g_ref[...] * bw + bf
    mm = jnp.dot(h, w1d_ref[...], preferred_element_type=jnp.float32) \
        + b1d_ref[...]
    m = jax.nn.silu(mm)
    rowid = row_off + i * _PB + lax.broadcasted_iota(jnp.int32, (_PB, 1), 0)
    m = jnp.where(rowid < E // 2, m, 0.0)
    m_ref[...] = m


def _msg_tc(row_off, bde3, bdo3, g128, wbw2, wbe2, bbe2, w1d, b1d):
    eh2 = EH // 2
    return pl.pallas_call(
        functools.partial(_msg_body, row_off),
        grid=(eh2 // _PB,),
        in_specs=[
            pl.BlockSpec((1, 1, _PB), lambda i: (i, 0, 0)),
            pl.BlockSpec((1, 1, _PB), lambda i: (i, 0, 0)),
            pl.BlockSpec((_PB, 128), lambda i: (i, 0)),
            pl.BlockSpec((2 * MAX_N, 128), lambda i: (0, 0)),
            pl.BlockSpec((2 * MAX_N, 128), lambda i: (0, 0)),
            pl.BlockSpec((1, 128), lambda i: (0, 0)),
            pl.BlockSpec((128, 128), lambda i: (0, 0)),
            pl.BlockSpec((1, 128), lambda i: (0, 0)),
        ],
        out_specs=pl.BlockSpec((_PB, 128), lambda i: (i, 0)),
        out_shape=jax.ShapeDtypeStruct((eh2, 128), jnp.float32),
    )(bde3, bdo3, g128, wbw2, wbe2, bbe2, w1d, b1d)


def _emb_body(t_ref, emb_ref, x_ref):
    t = t_ref[0, 0, :]
    oh = (t[:, None] == lax.broadcasted_iota(jnp.int32, (_NB, NELEM), 1)
          ).astype(jnp.float32)
    x_ref[...] = jnp.dot(oh, emb_ref[...], preferred_element_type=jnp.float32)


def _emb_tc(t3, emb):
    return pl.pallas_call(
        _emb_body,
        grid=(N // _NB,),
        in_specs=[
            pl.BlockSpec((1, 1, _NB), lambda i: (i, 0, 0)),
            pl.BlockSpec((NELEM, D), lambda i: (0, 0)),
        ],
        out_specs=pl.BlockSpec((_NB, D), lambda i: (i, 0)),
        out_shape=jax.ShapeDtypeStruct((N, D), jnp.float32),
    )(t3, emb)


def _upd_body(x_ref, agga_ref, aggb_ref, w2_ref, o_ref):
    a = agga_ref[:, :D] + aggb_ref[:, :D]
    y = jnp.dot(a, w2_ref[...], preferred_element_type=jnp.float32)
    o_ref[...] = x_ref[...] + y


def _upd_tc(x, agga, aggb, w2):
    return pl.pallas_call(
        _upd_body,
        grid=(N // _NB,),
        in_specs=[
            pl.BlockSpec((_NB, D), lambda i: (i, 0)),
            pl.BlockSpec((_NB, 128), lambda i: (i, 0)),
            pl.BlockSpec((_NB, 128), lambda i: (i, 0)),
            pl.BlockSpec((D, D), lambda i: (0, 0)),
        ],
        out_specs=pl.BlockSpec((_NB, D), lambda i: (i, 0)),
        out_shape=jax.ShapeDtypeStruct((N, D), jnp.float32),
    )(x, agga, aggb, w2)


def _final_body(x_ref, agga_ref, aggb_ref, w2_ref, wout_ref, bout_ref, o_ref):
    a = agga_ref[:, :D] + aggb_ref[:, :D]
    y = jnp.dot(a, w2_ref[...], preferred_element_type=jnp.float32)
    xf = x_ref[...] + y
    e = jnp.sum(xf * wout_ref[...], axis=1, keepdims=True) + bout_ref[0, 0]
    o_ref[...] = e


def _final_tc(x, agga, aggb, w2, wout2, bout2):
    return pl.pallas_call(
        _final_body,
        grid=(N // _NB,),
        in_specs=[
            pl.BlockSpec((_NB, D), lambda i: (i, 0)),
            pl.BlockSpec((_NB, 128), lambda i: (i, 0)),
            pl.BlockSpec((_NB, 128), lambda i: (i, 0)),
            pl.BlockSpec((D, D), lambda i: (0, 0)),
            pl.BlockSpec((1, D), lambda i: (0, 0)),
            pl.BlockSpec((1, 1), lambda i: (0, 0)),
        ],
        out_specs=pl.BlockSpec((_NB, 1), lambda i: (i, 0)),
        out_shape=jax.ShapeDtypeStruct((N, 1), jnp.float32),
    )(x, agga, aggb, w2, wout2, bout2)


def _block_diag2(w):
    z = jnp.zeros((w.shape[0] * 2, w.shape[1] * 2), jnp.float32)
    z = z.at[: w.shape[0], : w.shape[1]].set(w)
    return z.at[w.shape[0]:, w.shape[1]:].set(w)


def kernel(bond_dist, params, atom_types, edge_index):
    pad = E_PAD - E
    fill = (jnp.arange(pad, dtype=jnp.int32) * 997) % N
    src = jnp.concatenate([edge_index[0].astype(jnp.int32), fill])
    dst = jnp.concatenate([edge_index[1].astype(jnp.int32), fill])
    src2d = [src[:EH].reshape(EH // 128, 128), src[EH:].reshape(EH // 128, 128)]
    dst2d = [dst[:EH].reshape(EH // 128, 128), dst[EH:].reshape(EH // 128, 128)]
    bd_pad = jnp.concatenate([bond_dist, jnp.full((pad,), 0.5, jnp.float32)])
    eh2 = EH // 2
    bde3, bdo3 = [], []
    for h in range(2):
        bdp = bd_pad[h * EH:(h + 1) * EH].reshape(eh2, 2)
        bde3.append(bdp[:, 0].reshape(eh2 // _PB, 1, _PB))
        bdo3.append(bdp[:, 1].reshape(eh2 // _PB, 1, _PB))
    t3 = atom_types.astype(jnp.int32).reshape(N // _NB, 1, _NB)

    zeros_half = jnp.zeros((N, D // 2), jnp.float32)
    wbw2 = _block_diag2(params["W_bw"])
    wbe2 = _block_diag2(params["W_be"])
    bbe2 = jnp.concatenate([params["b_be"], params["b_be"]]).reshape(1, 128)
    wout2 = params["W_out"].reshape(1, D)
    bout2 = params["b_out"].reshape(1, 1).astype(jnp.float32)

    x = _emb_tc(t3, params["atom_emb"])
    row_offs = [0, eh2]
    for k, blk in enumerate(params["blocks"]):
        w1d = _block_diag2(blk["W1"])
        b1d = jnp.concatenate([blk["b1"], blk["b1"]]).reshape(1, 128)
        g = [_gather_sc(x, src2d[h]) for h in range(2)]
        m = [
            _msg_tc(row_offs[h], bde3[h], bdo3[h], g[h].reshape(eh2, 128),
                    wbw2, wbe2, bbe2, w1d, b1d)
            for h in range(2)
        ]
        agg = [_scatter_sc(m[h].reshape(EH, D), dst2d[h], zeros_half)
               for h in range(2)]
        if k < NBLOCKS - 1:
            x = _upd_tc(x, agg[0], agg[1], blk["W2"])
        else:
            out = _final_tc(x, agg[0], agg[1], blk["W2"], wout2, bout2)
    return out.reshape(N)


# speedup vs baseline: 5.7462x; 1.0111x over previous
import functools

import jax
import jax.numpy as jnp
from jax import lax
from jax.experimental import pallas as pl
from jax.experimental.pallas import tpu as pltpu
from jax.experimental.pallas import tpu_sc as plsc

N = 50000
E = 800000
D = 64
MAX_N = 9
CUTOFF = 5.0
NBLOCKS = 4
NELEM = 89

E_PAD = 819200
EH = E_PAD // 2
ECHUNK = 256
NSUB = ECHUNK // 128
NCORES = 2
NTILES = 16
NWORK = NCORES * NTILES
GCHUNK = 640
GSUB = GCHUNK // 128
FLUSH = 400
NFLUSH = N // FLUSH

_mesh = plsc.VectorSubcoreMesh(core_axis_name="c", subcore_axis_name="s")


def _make_gather(ne):
    chunks = ne // GCHUNK // NWORK

    @functools.partial(
        pl.kernel,
        out_type=jax.ShapeDtypeStruct((ne, D), jnp.float32),
        mesh=_mesh,
        scratch_types=[
            pltpu.VMEM((GSUB, 128), jnp.int32),
            pltpu.VMEM((GSUB, 128), jnp.int32),
            pltpu.VMEM((GCHUNK, D), jnp.float32),
            pltpu.VMEM((GCHUNK, D), jnp.float32),
            pltpu.SemaphoreType.DMA,
            pltpu.SemaphoreType.DMA,
            pltpu.SemaphoreType.DMA,
            pltpu.SemaphoreType.DMA,
            pltpu.SemaphoreType.DMA,
        ],
        compiler_params=pltpu.CompilerParams(use_tc_tiling_on_sc=False),
    )
    def gather_k(x_hbm, src2d_hbm, g_hbm, idx_v0, idx_v1, rows_v0, rows_v1,
                 sem_i0, sem_i1, sem_w0, sem_w1, sem_g):
        cid = lax.axis_index("c")
        sid = lax.axis_index("s")
        wid = sid * NCORES + cid
        base_chunk = wid * chunks
        idx_v = (idx_v0, idx_v1)
        rows_v = (rows_v0, rows_v1)
        sem_i = (sem_i0, sem_i1)
        sem_w = (sem_w0, sem_w1)

        def _fetch_idx(q, b):
            pltpu.async_copy(src2d_hbm.at[pl.ds(q * GSUB, GSUB)], idx_v[b],
                             sem_i[b])

        def _process(q, b, first):
            pltpu.make_async_copy(
                src2d_hbm.at[pl.ds(q * GSUB, GSUB)], idx_v[b], sem_i[b]
            ).wait()
            if not first:
                pltpu.make_async_copy(
                    rows_v[b], g_hbm.at[pl.ds(q * GCHUNK, GCHUNK)], sem_w[b]
                ).wait()
            descs = []
            for j in range(GSUB):
                descs.append(pltpu.async_copy(
                    x_hbm.at[idx_v[b].at[j]],
                    rows_v[b].at[pl.ds(j * 128, 128)],
                    sem_g,
                ))
            for d in descs:
                d.wait()
            pltpu.async_copy(rows_v[b], g_hbm.at[pl.ds(q * GCHUNK, GCHUNK)],
                             sem_w[b])

            @pl.when(q + 2 < base_chunk + chunks)
            def _():
                _fetch_idx(q + 2, b)

        for b in range(2):
            _fetch_idx(base_chunk + b, b)
        for b in range(2):
            _process(base_chunk + b, b, True)

        def body(i2, _):
            for b in range(2):
                _process(base_chunk + 2 + 2 * i2 + b, b, False)
            return 0

        lax.fori_loop(0, (chunks - 2) // 2, body, 0)
        for b in range(2):
            pltpu.make_async_copy(
                rows_v[b], g_hbm.at[pl.ds(base_chunk * GCHUNK, GCHUNK)],
                sem_w[b],
            ).wait()

    return gather_k


def _make_scatter(ne):
    chunks = ne // ECHUNK // NTILES

    @functools.partial(
        pl.kernel,
        out_type=jax.ShapeDtypeStruct((N, 128), jnp.float32),
        mesh=_mesh,
        scratch_types=[
            pltpu.VMEM_SHARED((N, D // 2), jnp.float32),
            pltpu.VMEM((NSUB, 128), jnp.int32),
            pltpu.VMEM((NSUB, 128), jnp.int32),
            pltpu.VMEM((ECHUNK, D // 2), jnp.float32),
            pltpu.VMEM((ECHUNK, D // 2), jnp.float32),
            pltpu.SemaphoreType.DMA,
            pltpu.SemaphoreType.DMA,
            pltpu.SemaphoreType.DMA,
            pltpu.SemaphoreType.DMA,
            pltpu.SemaphoreType.DMA,
        ],
        compiler_params=pltpu.CompilerParams(use_tc_tiling_on_sc=False),
    )
    def scatter_k(m_hbm, dst2d_hbm, zeros_hbm, agg_hbm, acc_sh,
                  idx_v0, idx_v1, mrow_v0, mrow_v1,
                  sem_i0, sem_i1, sem_m0, sem_m1, sem_s):
        cid = lax.axis_index("c")
        sid = lax.axis_index("s")
        c0 = cid * (D // 2)
        idx_v = (idx_v0, idx_v1)
        mrow_v = (mrow_v0, mrow_v1)
        sem_i = (sem_i0, sem_i1)
        sem_m = (sem_m0, sem_m1)

        def zero(k, _):
            q = sid + k * NTILES

            @pl.when(q < NFLUSH)
            def _():
                r0 = q * FLUSH
                pltpu.sync_copy(zeros_hbm.at[pl.ds(r0, FLUSH)],
                                acc_sh.at[pl.ds(r0, FLUSH)])
            return 0

        lax.fori_loop(0, (NFLUSH + NTILES - 1) // NTILES, zero, 0)
        plsc.subcore_barrier()

        base_chunk = sid * chunks

        def _fetch(q, b):
            pltpu.async_copy(dst2d_hbm.at[pl.ds(q * NSUB, NSUB)], idx_v[b],
                             sem_i[b])
            pltpu.async_copy(
                m_hbm.at[pl.ds(q * ECHUNK, ECHUNK), pl.ds(c0, D // 2)],
                mrow_v[b], sem_m[b])

        for b in range(2):
            _fetch(base_chunk + b, b)

        def body(i2, _):
            for b in range(2):
                i = 2 * i2 + b
                q = base_chunk + i
                pltpu.make_async_copy(
                    dst2d_hbm.at[pl.ds(q * NSUB, NSUB)], idx_v[b], sem_i[b]
                ).wait()
                pltpu.make_async_copy(
                    m_hbm.at[pl.ds(q * ECHUNK, ECHUNK), pl.ds(c0, D // 2)],
                    mrow_v[b], sem_m[b],
                ).wait()
                descs = []
                for j in range(NSUB):
                    descs.append(pltpu.async_copy(
                        mrow_v[b].at[pl.ds(j * 128, 128)],
                        acc_sh.at[idx_v[b].at[j]],
                        sem_s, add=True,
                    ))
                for d in descs:
                    d.wait()

                @pl.when(i + 2 < chunks)
                def _():
                    _fetch(q + 2, b)
            return 0

        lax.fori_loop(0, chunks // 2, body, 0)
        plsc.subcore_barrier()

        def flush(k, _):
            q = sid + k * NTILES

            @pl.when(q < NFLUSH)
            def _():
                r0 = q * FLUSH
                pltpu.sync_copy(
                    acc_sh.at[pl.ds(r0, FLUSH)],
                    agg_hbm.at[pl.ds(r0, FLUSH), pl.ds(c0, D // 2)])
            return 0

        lax.fori_loop(0, (NFLUSH + NTILES - 1) // NTILES, flush, 0)

    return scatter_k


_gather_sc = _make_gather(EH)
_scatter_sc = _make_scatter(EH)


_EB = 16384
_PB = _EB // 2
_NB = 1000


def _bessel_rows(bd):
    r = 0.5 + bd * (CUTOFF - 0.5)
    theta = r * (jnp.pi / CUTOFF)
    s1 = jnp.sin(theta)
    c1 = jnp.cos(theta)
    env = 0.5 * (c1 + 1.0)
    pref = jnp.sqrt(2.0 / CUTOFF) * env / r
    two_c1 = 2.0 * c1
    rows = [s1 * pref]
    s_prev, s_cur = jnp.zeros_like(s1), s1
    for _ in range(MAX_N - 1):
        s_prev, s_cur = s_cur, two_c1 * s_cur - s_prev
        rows.append(s_cur * pref)
    return rows


def _msg_body(row_off, bde_ref, bdo_ref, g_ref, wbw2_ref, wbe2_ref, bbe2_ref,
              w1d_ref, b1d_ref, m_ref):
    i = pl.program_id(0)
    rows = _bessel_rows(bde_ref[0, 0, :]) + _bessel_rows(bdo_ref[0, 0, :])
    smooth_t = jnp.concatenate([x[None, :] for x in rows], axis=0)
    cdims = (((0,), (0,)), ((), ()))
    bw = lax.dot_general(smooth_t, wbw2_ref[...], cdims,
                         preferred_element_type=jnp.float32)
    bf = jax.nn.silu(
        lax.dot_general(smooth_t, wbe2_ref[...], cdims,
                        preferred_element_type=jnp.float32)
        + bbe2_ref[...])
    h = g_ref[...] * bw + bf
    mm = jnp.dot(h, w1d_ref[...], preferred_element_type=jnp.float32) \
        + b1d_ref[...]
    m = jax.nn.silu(mm)
    rowid = row_off + i * _PB + lax.broadcasted_iota(jnp.int32, (_PB, 1), 0)
    m = jnp.where(rowid < E // 2, m, 0.0)
    m_ref[...] = m


def _msg_tc(row_off, bde3, bdo3, g128, wbw2, wbe2, bbe2, w1d, b1d):
    eh2 = EH // 2
    return pl.pallas_call(
        functools.partial(_msg_body, row_off),
        grid=(eh2 // _PB,),
        in_specs=[
            pl.BlockSpec((1, 1, _PB), lambda i: (i, 0, 0)),
            pl.BlockSpec((1, 1, _PB), lambda i: (i, 0, 0)),
            pl.BlockSpec((_PB, 128), lambda i: (i, 0)),
            pl.BlockSpec((2 * MAX_N, 128), lambda i: (0, 0)),
            pl.BlockSpec((2 * MAX_N, 128), lambda i: (0, 0)),
            pl.BlockSpec((1, 128), lambda i: (0, 0)),
            pl.BlockSpec((128, 128), lambda i: (0, 0)),
            pl.BlockSpec((1, 128), lambda i: (0, 0)),
        ],
        out_specs=pl.BlockSpec((_PB, 128), lambda i: (i, 0)),
        out_shape=jax.ShapeDtypeStruct((eh2, 128), jnp.float32),
    )(bde3, bdo3, g128, wbw2, wbe2, bbe2, w1d, b1d)


def _emb_body(t_ref, emb_ref, x_ref):
    t = t_ref[0, 0, :]
    oh = (t[:, None] == lax.broadcasted_iota(jnp.int32, (_NB, NELEM), 1)
          ).astype(jnp.float32)
    x_ref[...] = jnp.dot(oh, emb_ref[...], preferred_element_type=jnp.float32)


def _emb_tc(t3, emb):
    return pl.pallas_call(
        _emb_body,
        grid=(N // _NB,),
        in_specs=[
            pl.BlockSpec((1, 1, _NB), lambda i: (i, 0, 0)),
            pl.BlockSpec((NELEM, D), lambda i: (0, 0)),
        ],
        out_specs=pl.BlockSpec((_NB, D), lambda i: (i, 0)),
        out_shape=jax.ShapeDtypeStruct((N, D), jnp.float32),
    )(t3, emb)


def _upd_body(x_ref, agga_ref, aggb_ref, w2_ref, o_ref):
    a = agga_ref[:, :D] + aggb_ref[:, :D]
    y = jnp.dot(a, w2_ref[...], preferred_element_type=jnp.float32)
    o_ref[...] = x_ref[...] + y


def _upd_tc(x, agga, aggb, w2):
    return pl.pallas_call(
        _upd_body,
        grid=(N // _NB,),
        in_specs=[
            pl.BlockSpec((_NB, D), lambda i: (i, 0)),
            pl.BlockSpec((_NB, 128), lambda i: (i, 0)),
            pl.BlockSpec((_NB, 128), lambda i: (i, 0)),
            pl.BlockSpec((D, D), lambda i: (0, 0)),
        ],
        out_specs=pl.BlockSpec((_NB, D), lambda i: (i, 0)),
        out_shape=jax.ShapeDtypeStruct((N, D), jnp.float32),
    )(x, agga, aggb, w2)


def _final_body(x_ref, agga_ref, aggb_ref, w2_ref, wout_ref, bout_ref, o_ref):
    a = agga_ref[:, :D] + aggb_ref[:, :D]
    y = jnp.dot(a, w2_ref[...], preferred_element_type=jnp.float32)
    xf = x_ref[...] + y
    e = jnp.sum(xf * wout_ref[...], axis=1, keepdims=True) + bout_ref[0, 0]
    o_ref[...] = e


def _final_tc(x, agga, aggb, w2, wout2, bout2):
    return pl.pallas_call(
        _final_body,
        grid=(N // _NB,),
        in_specs=[
            pl.BlockSpec((_NB, D), lambda i: (i, 0)),
            pl.BlockSpec((_NB, 128), lambda i: (i, 0)),
            pl.BlockSpec((_NB, 128), lambda i: (i, 0)),
            pl.BlockSpec((D, D), lambda i: (0, 0)),
            pl.BlockSpec((1, D), lambda i: (0, 0)),
            pl.BlockSpec((1, 1), lambda i: (0, 0)),
        ],
        out_specs=pl.BlockSpec((_NB, 1), lambda i: (i, 0)),
        out_shape=jax.ShapeDtypeStruct((N, 1), jnp.float32),
    )(x, agga, aggb, w2, wout2, bout2)


def _block_diag2(w):
    z = jnp.zeros((w.shape[0] * 2, w.shape[1] * 2), jnp.float32)
    z = z.at[: w.shape[0], : w.shape[1]].set(w)
    return z.at[w.shape[0]:, w.shape[1]:].set(w)


def kernel(bond_dist, params, atom_types, edge_index):
    pad = E_PAD - E
    fill = (jnp.arange(pad, dtype=jnp.int32) * 997) % N
    src = jnp.concatenate([edge_index[0].astype(jnp.int32), fill])
    dst = jnp.concatenate([edge_index[1].astype(jnp.int32), fill])
    src2d = [src[:EH].reshape(EH // 128, 128), src[EH:].reshape(EH // 128, 128)]
    dst2d = [dst[:EH].reshape(EH // 128, 128), dst[EH:].reshape(EH // 128, 128)]
    bd_pad = jnp.concatenate([bond_dist, jnp.full((pad,), 0.5, jnp.float32)])
    eh2 = EH // 2
    bde3, bdo3 = [], []
    for h in range(2):
        bdp = bd_pad[h * EH:(h + 1) * EH].reshape(eh2, 2)
        bde3.append(bdp[:, 0].reshape(eh2 // _PB, 1, _PB))
        bdo3.append(bdp[:, 1].reshape(eh2 // _PB, 1, _PB))
    t3 = atom_types.astype(jnp.int32).reshape(N // _NB, 1, _NB)

    zeros_half = jnp.zeros((N, D // 2), jnp.float32)
    wbw2 = _block_diag2(params["W_bw"])
    wbe2 = _block_diag2(params["W_be"])
    bbe2 = jnp.concatenate([params["b_be"], params["b_be"]]).reshape(1, 128)
    wout2 = params["W_out"].reshape(1, D)
    bout2 = params["b_out"].reshape(1, 1).astype(jnp.float32)

    x = _emb_tc(t3, params["atom_emb"])
    row_offs = [0, eh2]
    for k, blk in enumerate(params["blocks"]):
        w1d = _block_diag2(blk["W1"])
        b1d = jnp.concatenate([blk["b1"], blk["b1"]]).reshape(1, 128)
        g = [_gather_sc(x, src2d[h]) for h in range(2)]
        m = [
            _msg_tc(row_offs[h], bde3[h], bdo3[h], g[h].reshape(eh2, 128),
                    wbw2, wbe2, bbe2, w1d, b1d)
            for h in range(2)
        ]
        agg = [_scatter_sc(m[h].reshape(EH, D), dst2d[h], zeros_half)
               for h in range(2)]
        if k < NBLOCKS - 1:
            x = _upd_tc(x, agg[0], agg[1], blk["W2"])
        else:
            out = _final_tc(x, agg[0], agg[1], blk["W2"], wout2, bout2)
    return out.reshape(N)
